# baseline (speedup 1.0000x reference)
import functools

import jax
import jax.numpy as jnp
from jax import lax
from jax.experimental import pallas as pl
from jax.experimental.pallas import tpu as pltpu
from jax.experimental.pallas import tpu_sc as plsc

_N = 10000
_E = 320000
_H = 128
_G = 64
_C = 10

_NSUB = 16
_CH = 128
_EPW = 20480
_NCH = _EPW // _CH
_EPAD = _NSUB * _EPW
_ACC = 10240
_TRASH = _N
_STRIPE = _ACC // _NSUB
_ZR = 64

def _zero_vmem(ref, rows, width):
  @pl.loop(0, rows)
  def _(r):
    @pl.loop(0, width // 16)
    def _(k):
      ref[r, pl.ds(k * 16, 16)] = jnp.zeros((16,), jnp.float32)


@functools.lru_cache(maxsize=None)
def _sc_kernels():
  mesh = plsc.VectorSubcoreMesh(core_axis_name="c", subcore_axis_name="s",
                                num_cores=2, num_subcores=_NSUB)

  @functools.partial(
      pl.kernel,
      out_type=jax.ShapeDtypeStruct((2, 2, _ACC, 64), jnp.float32),
      mesh=mesh,
      compiler_params=pltpu.CompilerParams(use_tc_tiling_on_sc=False),
      scratch_types=[
          pltpu.VMEM((2, _NCH, _CH), jnp.int32),
          pltpu.VMEM((_NCH, _CH), jnp.int32),
          pltpu.VMEM((2, _CH, 64), jnp.float32),
          pltpu.VMEM((_ZR, 64), jnp.float32),
          pltpu.VMEM_SHARED((_ACC, 64), jnp.float32),
          [pltpu.SemaphoreType.DMA] * 2,
          [pltpu.SemaphoreType.DMA] * 2,
      ],
  )
  def sc_scatter(y_hbm, srcx_hbm, dst_hbm, out_hbm,
                 src_v, dst_v, rows_v, z_v, acc_sh, gsems, ssems):
    c = lax.axis_index("c")
    s = lax.axis_index("s")
    pltpu.sync_copy(srcx_hbm.at[s], src_v)
    pltpu.sync_copy(dst_hbm.at[c, s], dst_v)
    _zero_vmem(z_v, _ZR, 64)
    base = s * _STRIPE

    for k in range(2):
      @pl.loop(0, _STRIPE // _ZR)
      def _(i):
        pltpu.sync_copy(z_v, acc_sh.at[pl.ds(base + i * _ZR, _ZR)])

      plsc.subcore_barrier()

      nring = 2
      for b in range(nring):
        pltpu.async_copy(y_hbm.at[src_v.at[k, b]], rows_v.at[b], gsems[b])

      @pl.loop(0, _NCH // nring)
      def _(jr):
        j = nring * jr
        for b in range(nring):
          pltpu.make_async_copy(y_hbm.at[src_v.at[k, 0]], rows_v.at[b],
                                gsems[b]).wait()
          pltpu.async_copy(rows_v.at[b], acc_sh.at[dst_v.at[j + b]],
                           ssems[b], add=True)
        for b in range(nring):
          @pl.when(j + b + nring < _NCH)
          def _():
            pltpu.make_async_copy(y_hbm.at[src_v.at[k, 0]], rows_v.at[b],
                                  ssems[b]).wait()
            pltpu.async_copy(y_hbm.at[src_v.at[k, j + b + nring]],
                             rows_v.at[b], gsems[b])

      for b in range(nring):
        pltpu.make_async_copy(y_hbm.at[src_v.at[k, 0]], rows_v.at[b],
                              ssems[b]).wait()

      plsc.subcore_barrier()
      pltpu.sync_copy(acc_sh.at[pl.ds(base, _STRIPE)],
                      out_hbm.at[c, k, pl.ds(base, _STRIPE)])

  @functools.partial(
      pl.kernel,
      out_type=jax.ShapeDtypeStruct((2, _ACC, 16), jnp.float32),
      mesh=mesh,
      compiler_params=pltpu.CompilerParams(use_tc_tiling_on_sc=False),
      scratch_types=[
          pltpu.VMEM((_NCH, _CH), jnp.int32),
          pltpu.VMEM((_CH, 16), jnp.float32),
          pltpu.VMEM((_ZR, 16), jnp.float32),
          pltpu.VMEM_SHARED((_ACC, 16), jnp.float32),
      ],
  )
  def sc_edge_stats(av_hbm, dst_hbm, out_hbm, dst_v, av_v, z_v, acc_sh):
    c = lax.axis_index("c")
    s = lax.axis_index("s")
    pltpu.sync_copy(dst_hbm.at[c, s], dst_v)
    _zero_vmem(z_v, _ZR, 16)
    base = s * _STRIPE

    @pl.loop(0, _STRIPE // _ZR)
    def _(i):
      pltpu.sync_copy(z_v, acc_sh.at[pl.ds(base + i * _ZR, _ZR)])

    plsc.subcore_barrier()

    @pl.loop(0, _NCH)
    def _(j):
      pltpu.sync_copy(av_hbm.at[s, j], av_v)
      pltpu.sync_copy(av_v, acc_sh.at[dst_v.at[j]], add=True)

    plsc.subcore_barrier()
    pltpu.sync_copy(acc_sh.at[pl.ds(base, _STRIPE)],
                    out_hbm.at[c, pl.ds(base, _STRIPE)])

  return sc_scatter, sc_edge_stats


def _sc_scatter(y4, srcx4, dst3):
  return _sc_kernels()[0](y4, srcx4, dst3)


def _sc_edge_stats(av3, dst3):
  return _sc_kernels()[1](av3, dst3)


_BN = 1000


def _full(shape):
  return pl.BlockSpec(shape, lambda i: (0,) * len(shape))


def _k1_body(x_ref, w1_ref, b1_ref, w2_ref, b2_ref,
             wr0_ref, wr1_ref, wroot_ref, bc_ref, y_ref, root_ref):
  x = x_ref[...]
  t = jnp.dot(x, w1_ref[...], preferred_element_type=jnp.float32) + b1_ref[...]
  h = jnp.dot(t, w2_ref[...], preferred_element_type=jnp.float32) + b2_ref[...]
  y_ref[0] = jnp.dot(h, wr0_ref[...], preferred_element_type=jnp.float32)
  y_ref[1] = jnp.dot(h, wr1_ref[...], preferred_element_type=jnp.float32)
  root_ref[...] = (jnp.dot(h, wroot_ref[...], preferred_element_type=jnp.float32)
                   + bc_ref[...])


def _tc_encode_l1(x, w1, b1, w2, b2, wr0, wr1, wroot, bc):
  d_in = x.shape[1]
  d_h = w2.shape[1]
  return pl.pallas_call(
      _k1_body,
      grid=(_N // _BN,),
      in_specs=[
          pl.BlockSpec((_BN, d_in), lambda i: (i, 0)),
          _full(w1.shape), _full(b1.shape), _full(w2.shape), _full(b2.shape),
          _full((d_h, _H)), _full((d_h, _H)), _full((d_h, _H)), _full(bc.shape),
      ],
      out_specs=[
          pl.BlockSpec((2, _BN, _H), lambda i: (0, i, 0)),
          pl.BlockSpec((_BN, _H), lambda i: (i, 0)),
      ],
      out_shape=[
          jax.ShapeDtypeStruct((2, _N, _H), jnp.float32),
          jax.ShapeDtypeStruct((_N, _H), jnp.float32),
      ],
  )(x, w1, b1, w2, b2, wr0, wr1, wroot, bc)


def _epilogue(root_ref, a0l_ref, a0r_ref, a1l_ref, a1r_ref, scn_ref, we_ref):
  s0 = scn_ref[:, 0:1]
  c0 = scn_ref[:, 1:2]
  s1 = scn_ref[:, 2:3]
  c1 = scn_ref[:, 3:4]
  we = we_ref[...]
  a0 = jnp.concatenate([a0l_ref[...], a0r_ref[...]], axis=1)
  a1 = jnp.concatenate([a1l_ref[...], a1r_ref[...]], axis=1)
  t0 = (a0 + s0 * we) / jnp.maximum(c0, 1.0)
  t1 = (a1 + s1 * we) / jnp.maximum(c1, 1.0)
  h = root_ref[...] + t0 + t1
  return jnp.where(h > 0.0, h, jnp.exp(jnp.minimum(h, 0.0)) - 1.0)


def _kmid_body(root_ref, a0l_ref, a0r_ref, a1l_ref, a1r_ref, scn_ref, we_ref,
               wr0_ref, wr1_ref, wroot_ref, bc_ref, y_ref, rootn_ref):
  h = _epilogue(root_ref, a0l_ref, a0r_ref, a1l_ref, a1r_ref, scn_ref, we_ref)
  y_ref[0] = jnp.dot(h, wr0_ref[...], preferred_element_type=jnp.float32)
  y_ref[1] = jnp.dot(h, wr1_ref[...], preferred_element_type=jnp.float32)
  rootn_ref[...] = (jnp.dot(h, wroot_ref[...],
                            preferred_element_type=jnp.float32) + bc_ref[...])


def _tc_mid(root, aggs, scn, we, wr0, wr1, wroot, bc):
  return pl.pallas_call(
      _kmid_body,
      grid=(_N // _BN,),
      in_specs=[
          pl.BlockSpec((_BN, _H), lambda i: (i, 0)),
          pl.BlockSpec((_BN, 64), lambda i: (i, 0)),
          pl.BlockSpec((_BN, 64), lambda i: (i, 0)),
          pl.BlockSpec((_BN, 64), lambda i: (i, 0)),
          pl.BlockSpec((_BN, 64), lambda i: (i, 0)),
          pl.BlockSpec((_BN, 8), lambda i: (i, 0)),
          _full(we.shape),
          _full((_H, _H)), _full((_H, _H)), _full((_H, _H)), _full(bc.shape),
      ],
      out_specs=[
          pl.BlockSpec((2, _BN, _H), lambda i: (0, i, 0)),
          pl.BlockSpec((_BN, _H), lambda i: (i, 0)),
      ],
      out_shape=[
          jax.ShapeDtypeStruct((2, _N, _H), jnp.float32),
          jax.ShapeDtypeStruct((_N, _H), jnp.float32),
      ],
  )(root, *aggs, scn, we, wr0, wr1, wroot, bc)


def _kfin_body(root_ref, a0l_ref, a0r_ref, a1l_ref, a1r_ref, scn_ref, we_ref,
               batch_ref, wl_ref, bl_ref, out_ref, p_acc, c_acc):
  i = pl.program_id(0)

  @pl.when(i == 0)
  def _():
    p_acc[...] = jnp.zeros_like(p_acc)
    c_acc[...] = jnp.zeros_like(c_acc)

  h = _epilogue(root_ref, a0l_ref, a0r_ref, a1l_ref, a1r_ref, scn_ref, we_ref)
  bf = batch_ref[...]
  gids = lax.broadcasted_iota(jnp.int32, (_BN, _G), 1).astype(jnp.float32)
  ob = (bf == gids).astype(jnp.float32)
  p_acc[...] += lax.dot_general(ob, h, (((0,), (0,)), ((), ())),
                                preferred_element_type=jnp.float32)
  c_acc[...] += jnp.sum(ob, axis=0)[:, None]

  @pl.when(i == _N // _BN - 1)
  def _():
    pooled = p_acc[...] / jnp.maximum(c_acc[...], 1.0)
    out_ref[...] = (jnp.dot(pooled, wl_ref[...],
                            preferred_element_type=jnp.float32) + bl_ref[...])


def _tc_final(root, aggs, scn, we, batchf, wl, bl):
  return pl.pallas_call(
      _kfin_body,
      grid=(_N // _BN,),
      in_specs=[
          pl.BlockSpec((_BN, _H), lambda i: (i, 0)),
          pl.BlockSpec((_BN, 64), lambda i: (i, 0)),
          pl.BlockSpec((_BN, 64), lambda i: (i, 0)),
          pl.BlockSpec((_BN, 64), lambda i: (i, 0)),
          pl.BlockSpec((_BN, 64), lambda i: (i, 0)),
          pl.BlockSpec((_BN, 8), lambda i: (i, 0)),
          _full(we.shape),
          pl.BlockSpec((_BN, 1), lambda i: (i, 0)),
          _full(wl.shape), _full(bl.shape),
      ],
      out_specs=pl.BlockSpec((_G, _C), lambda i: (0, 0)),
      out_shape=jax.ShapeDtypeStruct((_G, _C), jnp.float32),
      scratch_shapes=[
          pltpu.VMEM((_G, _H), jnp.float32),
          pltpu.VMEM((_G, 1), jnp.float32),
      ],
  )(root, *aggs, scn, we, batchf, wl, bl)


def kernel(x, edge_index, edge_attr, edge_type, batch,
           W1, b1, W2, b2,
           Wroot1, Wrel1, We1, bc1,
           Wroot2, Wrel2, We2, bc2,
           Wroot3, Wrel3, We3, bc3,
           Wroot4, Wrel4, We4, bc4,
           Wl, bl):
  src = edge_index[0]
  dst = edge_index[1]
  et = edge_type

  pad = _EPAD - _E
  srcx = jnp.pad(2 * (src + et * _N), (0, pad)).reshape(_NSUB, _NCH, _CH)
  srcx4 = jnp.stack([srcx, srcx + 1], axis=1)
  dst0 = jnp.where(et == 0, dst, _TRASH)
  dst1 = jnp.where(et == 1, dst, _TRASH)
  dst3 = jnp.stack([
      jnp.pad(dst0, (0, pad), constant_values=_TRASH),
      jnp.pad(dst1, (0, pad), constant_values=_TRASH),
  ]).reshape(2, _NSUB, _NCH, _CH)
  av = jnp.pad(jnp.concatenate(
      [edge_attr.astype(jnp.float32),
       jnp.ones((_E, 1), jnp.float32)], axis=1), ((0, pad), (0, 14)))
  av3 = av.reshape(_NSUB, _NCH, _CH, 16)

  stats = _sc_edge_stats(av3, dst3)
  scn = jnp.concatenate([
      stats[0, :_N, 0:2], stats[1, :_N, 0:2],
      jnp.zeros((_N, 4), jnp.float32)], axis=1)

  x = x.astype(jnp.float32)
  y, root = _tc_encode_l1(x, W1, b1.reshape(1, -1), W2, b2.reshape(1, -1),
                          Wrel1[0], Wrel1[1], Wroot1, bc1.reshape(1, -1))

  def agg_slabs(y):
    agg = _sc_scatter(y.reshape(4 * _N, 64), srcx4, dst3)
    return (agg[0, 0, :_N], agg[0, 1, :_N], agg[1, 0, :_N], agg[1, 1, :_N])

  for Wroot, Wrel, We, bc in ((Wroot2, Wrel2, We1, bc2),
                              (Wroot3, Wrel3, We2, bc3),
                              (Wroot4, Wrel4, We3, bc4)):
    y, root = _tc_mid(root, agg_slabs(y), scn, We.reshape(1, -1),
                      Wrel[0], Wrel[1], Wroot, bc.reshape(1, -1))

  batchf = batch.astype(jnp.float32).reshape(_N, 1)
  return _tc_final(root, agg_slabs(y), scn, We4.reshape(1, -1),
                   batchf, Wl, bl.reshape(1, -1))


# speedup vs baseline: 2.2223x; 2.2223x over previous
import functools

import jax
import jax.numpy as jnp
from jax import lax
from jax.experimental import pallas as pl
from jax.experimental.pallas import tpu as pltpu
from jax.experimental.pallas import tpu_sc as plsc

_N = 10000
_E = 320000
_H = 128
_G = 64
_C = 10

_NSUB = 16
_CH = 128
_EPW = 20480
_NCH = _EPW // _CH
_EPAD = _NSUB * _EPW
_NQ = 4
_QR = 2560
_ACCQ = 2688
_QTRASH = 2600
_QSTR = _QR // _NSUB
_CB = 20736

_SACC = 10240
_STRASH = _N
_SSTR = _SACC // _NSUB
_ZR = 64


def _zero_vmem(ref, rows, width):
  @pl.loop(0, rows)
  def _(r):
    @pl.loop(0, width // 16)
    def _(k):
      ref[r, pl.ds(k * 16, 16)] = jnp.zeros((16,), jnp.float32)


@functools.lru_cache(maxsize=None)
def _sc_kernels():
  mesh = plsc.VectorSubcoreMesh(core_axis_name="c", subcore_axis_name="s",
                                num_cores=2, num_subcores=_NSUB)
  params = pltpu.CompilerParams(use_tc_tiling_on_sc=False,
                                needs_layout_passes=False)

  @functools.partial(
      pl.kernel,
      out_type=[
          jax.ShapeDtypeStruct((2, _NQ, _NSUB, _NCH, _CH), jnp.int32),
          jax.ShapeDtypeStruct((2, _NQ, _NSUB, _NCH, _CH), jnp.int32),
          jax.ShapeDtypeStruct((2, _NQ, _NSUB, 16), jnp.int32),
      ],
      mesh=mesh,
      compiler_params=params,
      scratch_types=[
          pltpu.VMEM((_EPW,), jnp.int32),
          pltpu.VMEM((_EPW,), jnp.int32),
          pltpu.VMEM((_EPW,), jnp.int32),
          pltpu.VMEM((_NCH + 3, _CH), jnp.int32),
          pltpu.VMEM((_NCH + 3, _CH), jnp.int32),
          pltpu.VMEM((16,), jnp.int32),
      ],
  )
  def sc_compact(src_hbm, dst_hbm, typ_hbm, srcb_hbm, dstb_hbm, nch_hbm,
                 src_sl, dst_sl, typ_sl, sbuf, dbuf, cnt_v):
    c = lax.axis_index("c")
    s = lax.axis_index("s")
    pltpu.sync_copy(src_hbm.at[s], src_sl)
    pltpu.sync_copy(dst_hbm.at[s], dst_sl)
    pltpu.sync_copy(typ_hbm.at[s], typ_sl)
    lanes = lax.iota(jnp.int32, 16)
    srow = c * _N

    for q in range(_NQ):
      @plsc.parallel_loop(0, _EPW // 16, carry=jnp.int32(0))
      def o(i, o_in):
        li = 16 * i + lanes
        sv = plsc.load_gather(src_sl, [li])
        dv = plsc.load_gather(dst_sl, [li])
        tv = plsc.load_gather(typ_sl, [li])
        m = (tv == c) & (dv >= _QR * q) & (dv < _QR * (q + 1))
        mi = m.astype(jnp.int32)
        idx = o_in + plsc.cumsum(mi) - 1
        plsc.store_scatter(sbuf, [idx >> 7, idx & 127], sv + srow, mask=m)
        plsc.store_scatter(dbuf, [idx >> 7, idx & 127], dv - _QR * q, mask=m)
        return o_in + jnp.sum(mi)

      ones = jnp.full((16,), True)
      for t in range(_CH // 16 + 1):
        pidx = o + lanes + 16 * t
        plsc.store_scatter(sbuf, [pidx >> 7, pidx & 127],
                           jnp.full((16,), 0, jnp.int32) + srow, mask=ones)
        plsc.store_scatter(dbuf, [pidx >> 7, pidx & 127],
                           jnp.full((16,), _QTRASH, jnp.int32), mask=ones)
      nch = (o + _CH - 1) // _CH

      pltpu.sync_copy(sbuf.at[pl.ds(0, _NCH)], srcb_hbm.at[c, q, s])
      pltpu.sync_copy(dbuf.at[pl.ds(0, _NCH)], dstb_hbm.at[c, q, s])
      cnt_v[...] = jnp.zeros((16,), jnp.int32) + nch
      pltpu.sync_copy(cnt_v, nch_hbm.at[c, q, s])

  @functools.partial(
      pl.kernel,
      out_type=jax.ShapeDtypeStruct((2, _NQ, _QR, 128), jnp.float32),
      mesh=mesh,
      compiler_params=params,
      scratch_types=[
          pltpu.VMEM((_NCH, _CH), jnp.int32),
          pltpu.VMEM((_NCH, _CH), jnp.int32),
          pltpu.VMEM((2, _CH, 128), jnp.float32),
          pltpu.VMEM((_QSTR // 2, 128), jnp.float32),
          pltpu.VMEM((16,), jnp.int32),
          pltpu.VMEM_SHARED((_ACCQ, 128), jnp.float32),
          [pltpu.SemaphoreType.DMA] * 2,
          [pltpu.SemaphoreType.DMA] * 2,
      ],
  )
  def sc_scatter(y_hbm, srcb_hbm, dstb_hbm, nch_hbm, out_hbm,
                 src_v, dst_v, rows_v, z_v, cnt_v, acc_sh, gsems, ssems):
    c = lax.axis_index("c")
    s = lax.axis_index("s")
    _zero_vmem(z_v, _QSTR // 2, 128)
    base = s * _QSTR

    for q in range(_NQ):
      pltpu.sync_copy(nch_hbm.at[c, q, s], cnt_v)
      nch = jnp.max(cnt_v[...])
      pltpu.sync_copy(srcb_hbm.at[c, q, s], src_v)
      pltpu.sync_copy(dstb_hbm.at[c, q, s], dst_v)
      for hh in range(2):
        pltpu.sync_copy(z_v, acc_sh.at[pl.ds(base + hh * (_QSTR // 2),
                                             _QSTR // 2)])
      plsc.subcore_barrier()

      for b in range(2):
        @pl.when(b < nch)
        def _():
          pltpu.async_copy(y_hbm.at[src_v.at[b]], rows_v.at[b], gsems[b])

      @pl.loop(0, (nch + 1) // 2)
      def _(jr):
        j = 2 * jr
        for b in range(2):
          @pl.when(j + b < nch)
          def _():
            pltpu.make_async_copy(y_hbm.at[src_v.at[0]], rows_v.at[b],
                                  gsems[b]).wait()
            pltpu.async_copy(rows_v.at[b], acc_sh.at[dst_v.at[j + b]],
                             ssems[b], add=True)
        for b in range(2):
          @pl.when(j + b + 2 < nch)
          def _():
            pltpu.make_async_copy(y_hbm.at[src_v.at[0]], rows_v.at[b],
                                  ssems[b]).wait()
            pltpu.async_copy(y_hbm.at[src_v.at[j + b + 2]], rows_v.at[b],
                             gsems[b])

      for b in range(2):
        @pl.when(b < jnp.minimum(nch, 2))
        def _():
          pltpu.make_async_copy(y_hbm.at[src_v.at[0]], rows_v.at[b],
                                ssems[b]).wait()

      plsc.subcore_barrier()
      pltpu.sync_copy(acc_sh.at[pl.ds(base, _QSTR)],
                      out_hbm.at[c, q, pl.ds(base, _QSTR)])

  @functools.partial(
      pl.kernel,
      out_type=jax.ShapeDtypeStruct((2, _SACC, 16), jnp.float32),
      mesh=mesh,
      compiler_params=params,
      scratch_types=[
          pltpu.VMEM((_NCH, _CH), jnp.int32),
          pltpu.VMEM((_CH, 16), jnp.float32),
          pltpu.VMEM((_ZR, 16), jnp.float32),
          pltpu.VMEM_SHARED((_SACC, 16), jnp.float32),
      ],
  )
  def sc_edge_stats(av_hbm, dst_hbm, out_hbm, dst_v, av_v, z_v, acc_sh):
    c = lax.axis_index("c")
    s = lax.axis_index("s")
    pltpu.sync_copy(dst_hbm.at[c, s], dst_v)
    _zero_vmem(z_v, _ZR, 16)
    base = s * _SSTR

    @pl.loop(0, _SSTR // _ZR)
    def _(i):
      pltpu.sync_copy(z_v, acc_sh.at[pl.ds(base + i * _ZR, _ZR)])

    plsc.subcore_barrier()

    @pl.loop(0, _NCH)
    def _(j):
      pltpu.sync_copy(av_hbm.at[s, j], av_v)
      pltpu.sync_copy(av_v, acc_sh.at[dst_v.at[j]], add=True)

    plsc.subcore_barrier()
    pltpu.sync_copy(acc_sh.at[pl.ds(base, _SSTR)],
                    out_hbm.at[c, pl.ds(base, _SSTR)])

  return sc_compact, sc_scatter, sc_edge_stats


def _sc_compact(src3, dst3p, typ3):
  return _sc_kernels()[0](src3, dst3p, typ3)


def _sc_scatter(y, srcb, dstb, nchb):
  return _sc_kernels()[1](y, srcb, dstb, nchb)


def _sc_edge_stats(av3, dst3):
  return _sc_kernels()[2](av3, dst3)


_BN = 1000


def _full(shape):
  return pl.BlockSpec(shape, lambda i: (0,) * len(shape))


def _k1_body(x_ref, w1_ref, b1_ref, w2_ref, b2_ref,
             wr0_ref, wr1_ref, wroot_ref, bc_ref, y_ref, root_ref):
  x = x_ref[...]
  t = jnp.dot(x, w1_ref[...], preferred_element_type=jnp.float32) + b1_ref[...]
  h = jnp.dot(t, w2_ref[...], preferred_element_type=jnp.float32) + b2_ref[...]
  y_ref[0] = jnp.dot(h, wr0_ref[...], preferred_element_type=jnp.float32)
  y_ref[1] = jnp.dot(h, wr1_ref[...], preferred_element_type=jnp.float32)
  root_ref[...] = (jnp.dot(h, wroot_ref[...], preferred_element_type=jnp.float32)
                   + bc_ref[...])


def _tc_encode_l1(x, w1, b1, w2, b2, wr0, wr1, wroot, bc):
  d_in = x.shape[1]
  d_h = w2.shape[1]
  return pl.pallas_call(
      _k1_body,
      grid=(_N // _BN,),
      in_specs=[
          pl.BlockSpec((_BN, d_in), lambda i: (i, 0)),
          _full(w1.shape), _full(b1.shape), _full(w2.shape), _full(b2.shape),
          _full((d_h, _H)), _full((d_h, _H)), _full((d_h, _H)), _full(bc.shape),
      ],
      out_specs=[
          pl.BlockSpec((2, _BN, _H), lambda i: (0, i, 0)),
          pl.BlockSpec((_BN, _H), lambda i: (i, 0)),
      ],
      out_shape=[
          jax.ShapeDtypeStruct((2, _N, _H), jnp.float32),
          jax.ShapeDtypeStruct((_N, _H), jnp.float32),
      ],
  )(x, w1, b1, w2, b2, wr0, wr1, wroot, bc)


def _epilogue(root_ref, a0_ref, a1_ref, scn_ref, we_ref):
  s0 = scn_ref[:, 0:1]
  c0 = scn_ref[:, 1:2]
  s1 = scn_ref[:, 2:3]
  c1 = scn_ref[:, 3:4]
  we = we_ref[...]
  t0 = (a0_ref[...] + s0 * we) / jnp.maximum(c0, 1.0)
  t1 = (a1_ref[...] + s1 * we) / jnp.maximum(c1, 1.0)
  h = root_ref[...] + t0 + t1
  return jnp.where(h > 0.0, h, jnp.exp(jnp.minimum(h, 0.0)) - 1.0)


def _kmid_body(root_ref, a0_ref, a1_ref, scn_ref, we_ref,
               wr0_ref, wr1_ref, wroot_ref, bc_ref, y_ref, rootn_ref):
  h = _epilogue(root_ref, a0_ref, a1_ref, scn_ref, we_ref)
  y_ref[0] = jnp.dot(h, wr0_ref[...], preferred_element_type=jnp.float32)
  y_ref[1] = jnp.dot(h, wr1_ref[...], preferred_element_type=jnp.float32)
  rootn_ref[...] = (jnp.dot(h, wroot_ref[...],
                            preferred_element_type=jnp.float32) + bc_ref[...])


def _tc_mid(root, a0, a1, scn, we, wr0, wr1, wroot, bc):
  return pl.pallas_call(
      _kmid_body,
      grid=(_N // _BN,),
      in_specs=[
          pl.BlockSpec((_BN, _H), lambda i: (i, 0)),
          pl.BlockSpec((_BN, _H), lambda i: (i, 0)),
          pl.BlockSpec((_BN, _H), lambda i: (i, 0)),
          pl.BlockSpec((_BN, 8), lambda i: (i, 0)),
          _full(we.shape),
          _full((_H, _H)), _full((_H, _H)), _full((_H, _H)), _full(bc.shape),
      ],
      out_specs=[
          pl.BlockSpec((2, _BN, _H), lambda i: (0, i, 0)),
          pl.BlockSpec((_BN, _H), lambda i: (i, 0)),
      ],
      out_shape=[
          jax.ShapeDtypeStruct((2, _N, _H), jnp.float32),
          jax.ShapeDtypeStruct((_N, _H), jnp.float32),
      ],
  )(root, a0, a1, scn, we, wr0, wr1, wroot, bc)


def _kfin_body(root_ref, a0_ref, a1_ref, scn_ref, we_ref,
               batch_ref, wl_ref, bl_ref, out_ref, p_acc, c_acc):
  i = pl.program_id(0)

  @pl.when(i == 0)
  def _():
    p_acc[...] = jnp.zeros_like(p_acc)
    c_acc[...] = jnp.zeros_like(c_acc)

  h = _epilogue(root_ref, a0_ref, a1_ref, scn_ref, we_ref)
  bf = batch_ref[...]
  gids = lax.broadcasted_iota(jnp.int32, (_BN, _G), 1).astype(jnp.float32)
  ob = (bf == gids).astype(jnp.float32)
  p_acc[...] += lax.dot_general(ob, h, (((0,), (0,)), ((), ())),
                                preferred_element_type=jnp.float32)
  c_acc[...] += jnp.sum(ob, axis=0)[:, None]

  @pl.when(i == _N // _BN - 1)
  def _():
    pooled = p_acc[...] / jnp.maximum(c_acc[...], 1.0)
    out_ref[...] = (jnp.dot(pooled, wl_ref[...],
                            preferred_element_type=jnp.float32) + bl_ref[...])


def _tc_final(root, a0, a1, scn, we, batchf, wl, bl):
  return pl.pallas_call(
      _kfin_body,
      grid=(_N // _BN,),
      in_specs=[
          pl.BlockSpec((_BN, _H), lambda i: (i, 0)),
          pl.BlockSpec((_BN, _H), lambda i: (i, 0)),
          pl.BlockSpec((_BN, _H), lambda i: (i, 0)),
          pl.BlockSpec((_BN, 8), lambda i: (i, 0)),
          _full(we.shape),
          pl.BlockSpec((_BN, 1), lambda i: (i, 0)),
          _full(wl.shape), _full(bl.shape),
      ],
      out_specs=pl.BlockSpec((_G, _C), lambda i: (0, 0)),
      out_shape=jax.ShapeDtypeStruct((_G, _C), jnp.float32),
      scratch_shapes=[
          pltpu.VMEM((_G, _H), jnp.float32),
          pltpu.VMEM((_G, 1), jnp.float32),
      ],
  )(root, a0, a1, scn, we, batchf, wl, bl)


def kernel(x, edge_index, edge_attr, edge_type, batch,
           W1, b1, W2, b2,
           Wroot1, Wrel1, We1, bc1,
           Wroot2, Wrel2, We2, bc2,
           Wroot3, Wrel3, We3, bc3,
           Wroot4, Wrel4, We4, bc4,
           Wl, bl):
  src = edge_index[0]
  dst = edge_index[1]
  et = edge_type

  pad = _EPAD - _E
  src3 = jnp.pad(src, (0, pad)).reshape(_NSUB, _EPW)
  dst3p = jnp.pad(dst, (0, pad)).reshape(_NSUB, _EPW)
  typ3 = jnp.pad(et, (0, pad), constant_values=2).reshape(_NSUB, _EPW)

  srcb, dstb, nchb = _sc_compact(src3, dst3p, typ3)

  dst0 = jnp.where(et == 0, dst, _STRASH)
  dst1 = jnp.where(et == 1, dst, _STRASH)
  dst3 = jnp.stack([
      jnp.pad(dst0, (0, pad), constant_values=_STRASH),
      jnp.pad(dst1, (0, pad), constant_values=_STRASH),
  ]).reshape(2, _NSUB, _NCH, _CH)
  av = jnp.pad(jnp.concatenate(
      [edge_attr.astype(jnp.float32),
       jnp.ones((_E, 1), jnp.float32)], axis=1), ((0, pad), (0, 14)))
  av3 = av.reshape(_NSUB, _NCH, _CH, 16)

  stats = _sc_edge_stats(av3, dst3)
  scn = jnp.concatenate([
      stats[0, :_N, 0:2], stats[1, :_N, 0:2],
      jnp.zeros((_N, 4), jnp.float32)], axis=1)

  x = x.astype(jnp.float32)
  y, root = _tc_encode_l1(x, W1, b1.reshape(1, -1), W2, b2.reshape(1, -1),
                          Wrel1[0], Wrel1[1], Wroot1, bc1.reshape(1, -1))

  def aggs(y):
    out = _sc_scatter(y.reshape(2 * _N, _H), srcb, dstb, nchb)
    a = out.reshape(2, _NQ * _QR, 128)
    return a[0, :_N], a[1, :_N]

  for Wroot, Wrel, We, bc in ((Wroot2, Wrel2, We1, bc2),
                              (Wroot3, Wrel3, We2, bc3),
                              (Wroot4, Wrel4, We3, bc4)):
    a0, a1 = aggs(y)
    y, root = _tc_mid(root, a0, a1, scn, We.reshape(1, -1),
                      Wrel[0], Wrel[1], Wroot, bc.reshape(1, -1))

  a0, a1 = aggs(y)
  batchf = batch.astype(jnp.float32).reshape(_N, 1)
  return _tc_final(root, a0, a1, scn, We4.reshape(1, -1),
                   batchf, Wl, bl.reshape(1, -1))


# speedup vs baseline: 2.3751x; 1.0688x over previous
import functools

import jax
import jax.numpy as jnp
from jax import lax
from jax.experimental import pallas as pl
from jax.experimental.pallas import tpu as pltpu
from jax.experimental.pallas import tpu_sc as plsc

_N = 10000
_E = 320000
_H = 128
_G = 64
_C = 10

_NSUB = 16
_CH = 128
_EPW = 20480
_NCH = _EPW // _CH
_EPAD = _NSUB * _EPW
_NQ = 4
_QR = 2560
_ACCQ = 2688
_QTRASH = 2600
_QSTR = _QR // _NSUB
_CB = 20736

_SACC = 10240
_STRASH = _N
_SSTR = _SACC // _NSUB
_ZR = 64


def _zero_vmem(ref, rows, width):
  @pl.loop(0, rows)
  def _(r):
    @pl.loop(0, width // 16)
    def _(k):
      ref[r, pl.ds(k * 16, 16)] = jnp.zeros((16,), jnp.float32)


@functools.lru_cache(maxsize=None)
def _sc_kernels():
  mesh = plsc.VectorSubcoreMesh(core_axis_name="c", subcore_axis_name="s",
                                num_cores=2, num_subcores=_NSUB)
  params = pltpu.CompilerParams(use_tc_tiling_on_sc=False,
                                needs_layout_passes=False)

  @functools.partial(
      pl.kernel,
      out_type=[
          jax.ShapeDtypeStruct((2, _NQ, _NSUB, _NCH, _CH), jnp.int32),
          jax.ShapeDtypeStruct((2, _NQ, _NSUB, _NCH, _CH), jnp.int32),
          jax.ShapeDtypeStruct((2, _NQ, _NSUB, 16), jnp.int32),
      ],
      mesh=mesh,
      compiler_params=params,
      scratch_types=[
          pltpu.VMEM((_EPW,), jnp.int32),
          pltpu.VMEM((_EPW,), jnp.int32),
          pltpu.VMEM((_EPW,), jnp.int32),
          pltpu.VMEM((_NCH + 3, _CH), jnp.int32),
          pltpu.VMEM((_NCH + 3, _CH), jnp.int32),
          pltpu.VMEM((16,), jnp.int32),
      ],
  )
  def sc_compact(src_hbm, dst_hbm, typ_hbm, srcb_hbm, dstb_hbm, nch_hbm,
                 src_sl, dst_sl, typ_sl, sbuf, dbuf, cnt_v):
    c = lax.axis_index("c")
    s = lax.axis_index("s")
    pltpu.sync_copy(src_hbm.at[s], src_sl)
    pltpu.sync_copy(dst_hbm.at[s], dst_sl)
    pltpu.sync_copy(typ_hbm.at[s], typ_sl)
    lanes = lax.iota(jnp.int32, 16)
    srow = c * _N

    for q in range(_NQ):
      @plsc.parallel_loop(0, _EPW // 16, carry=jnp.int32(0))
      def o(i, o_in):
        li = 16 * i + lanes
        sv = plsc.load_gather(src_sl, [li])
        dv = plsc.load_gather(dst_sl, [li])
        tv = plsc.load_gather(typ_sl, [li])
        m = (tv == c) & (dv >= _QR * q) & (dv < _QR * (q + 1))
        mi = m.astype(jnp.int32)
        idx = o_in + plsc.cumsum(mi) - 1
        plsc.store_scatter(sbuf, [idx >> 7, idx & 127], sv + srow, mask=m)
        plsc.store_scatter(dbuf, [idx >> 7, idx & 127], dv - _QR * q, mask=m)
        return o_in + jnp.sum(mi)

      ones = jnp.full((16,), True)
      for t in range(_CH // 16 + 1):
        pidx = o + lanes + 16 * t
        plsc.store_scatter(sbuf, [pidx >> 7, pidx & 127],
                           jnp.full((16,), 0, jnp.int32) + srow, mask=ones)
        plsc.store_scatter(dbuf, [pidx >> 7, pidx & 127],
                           jnp.full((16,), _QTRASH, jnp.int32), mask=ones)
      nch = (o + _CH - 1) // _CH

      pltpu.sync_copy(sbuf.at[pl.ds(0, _NCH)], srcb_hbm.at[c, q, s])
      pltpu.sync_copy(dbuf.at[pl.ds(0, _NCH)], dstb_hbm.at[c, q, s])
      cnt_v[...] = jnp.zeros((16,), jnp.int32) + nch
      pltpu.sync_copy(cnt_v, nch_hbm.at[c, q, s])

  @functools.partial(
      pl.kernel,
      out_type=jax.ShapeDtypeStruct((2, _NQ, _QR, 128), jnp.float32),
      mesh=mesh,
      compiler_params=params,
      scratch_types=[
          pltpu.VMEM((_NCH, _CH), jnp.int32),
          pltpu.VMEM((_NCH, _CH), jnp.int32),
          pltpu.VMEM((3, _CH, 128), jnp.float32),
          pltpu.VMEM((_QSTR // 2, 128), jnp.float32),
          pltpu.VMEM((16,), jnp.int32),
          pltpu.VMEM_SHARED((_ACCQ, 128), jnp.float32),
          [pltpu.SemaphoreType.DMA] * 3,
          [pltpu.SemaphoreType.DMA] * 3,
      ],
  )
  def sc_scatter(y_hbm, srcb_hbm, dstb_hbm, nch_hbm, out_hbm,
                 src_v, dst_v, rows_v, z_v, cnt_v, acc_sh, gsems, ssems):
    c = lax.axis_index("c")
    s = lax.axis_index("s")
    _zero_vmem(z_v, _QSTR // 2, 128)
    base = s * _QSTR

    for q in range(_NQ):
      pltpu.sync_copy(nch_hbm.at[c, q, s], cnt_v)
      nch = jnp.max(cnt_v[...])
      for p in range(4):
        @pl.when(p * 40 < nch)
        def _():
          pltpu.sync_copy(srcb_hbm.at[c, q, s, pl.ds(p * 40, 40)],
                          src_v.at[pl.ds(p * 40, 40)])
          pltpu.sync_copy(dstb_hbm.at[c, q, s, pl.ds(p * 40, 40)],
                          dst_v.at[pl.ds(p * 40, 40)])
      for hh in range(2):
        pltpu.sync_copy(z_v, acc_sh.at[pl.ds(base + hh * (_QSTR // 2),
                                             _QSTR // 2)])
      plsc.subcore_barrier()

      for b in range(3):
        @pl.when(b < nch)
        def _():
          pltpu.async_copy(y_hbm.at[src_v.at[b]], rows_v.at[b], gsems[b])

      @pl.loop(0, (nch + 2) // 3)
      def _(jr):
        j = 3 * jr
        for b in range(3):
          @pl.when(j + b < nch)
          def _():
            pltpu.make_async_copy(y_hbm.at[src_v.at[0]], rows_v.at[b],
                                  gsems[b]).wait()
            pltpu.async_copy(rows_v.at[b], acc_sh.at[dst_v.at[j + b]],
                             ssems[b], add=True)
        for b in range(3):
          @pl.when(j + b + 3 < nch)
          def _():
            pltpu.make_async_copy(y_hbm.at[src_v.at[0]], rows_v.at[b],
                                  ssems[b]).wait()
            pltpu.async_copy(y_hbm.at[src_v.at[j + b + 3]], rows_v.at[b],
                             gsems[b])

      for b in range(3):
        @pl.when(b < jnp.minimum(nch, 3))
        def _():
          pltpu.make_async_copy(y_hbm.at[src_v.at[0]], rows_v.at[b],
                                ssems[b]).wait()

      plsc.subcore_barrier()
      pltpu.sync_copy(acc_sh.at[pl.ds(base, _QSTR)],
                      out_hbm.at[c, q, pl.ds(base, _QSTR)])

  @functools.partial(
      pl.kernel,
      out_type=jax.ShapeDtypeStruct((2, _SACC, 16), jnp.float32),
      mesh=mesh,
      compiler_params=params,
      scratch_types=[
          pltpu.VMEM((_NCH, _CH), jnp.int32),
          pltpu.VMEM((_CH, 16), jnp.float32),
          pltpu.VMEM((_ZR, 16), jnp.float32),
          pltpu.VMEM_SHARED((_SACC, 16), jnp.float32),
      ],
  )
  def sc_edge_stats(av_hbm, dst_hbm, out_hbm, dst_v, av_v, z_v, acc_sh):
    c = lax.axis_index("c")
    s = lax.axis_index("s")
    pltpu.sync_copy(dst_hbm.at[c, s], dst_v)
    _zero_vmem(z_v, _ZR, 16)
    base = s * _SSTR

    @pl.loop(0, _SSTR // _ZR)
    def _(i):
      pltpu.sync_copy(z_v, acc_sh.at[pl.ds(base + i * _ZR, _ZR)])

    plsc.subcore_barrier()

    @pl.loop(0, _NCH)
    def _(j):
      pltpu.sync_copy(av_hbm.at[s, j], av_v)
      pltpu.sync_copy(av_v, acc_sh.at[dst_v.at[j]], add=True)

    plsc.subcore_barrier()
    pltpu.sync_copy(acc_sh.at[pl.ds(base, _SSTR)],
                    out_hbm.at[c, pl.ds(base, _SSTR)])

  return sc_compact, sc_scatter, sc_edge_stats


def _sc_compact(src3, dst3p, typ3):
  return _sc_kernels()[0](src3, dst3p, typ3)


def _sc_scatter(y, srcb, dstb, nchb):
  return _sc_kernels()[1](y, srcb, dstb, nchb)


def _sc_edge_stats(av3, dst3):
  return _sc_kernels()[2](av3, dst3)


_BN = 1000


def _full(shape):
  return pl.BlockSpec(shape, lambda i: (0,) * len(shape))


def _k1_body(x_ref, w1_ref, b1_ref, w2_ref, b2_ref,
             wr0_ref, wr1_ref, wroot_ref, bc_ref, y_ref, root_ref):
  x = x_ref[...]
  t = jnp.dot(x, w1_ref[...], preferred_element_type=jnp.float32) + b1_ref[...]
  h = jnp.dot(t, w2_ref[...], preferred_element_type=jnp.float32) + b2_ref[...]
  y_ref[0] = jnp.dot(h, wr0_ref[...], preferred_element_type=jnp.float32)
  y_ref[1] = jnp.dot(h, wr1_ref[...], preferred_element_type=jnp.float32)
  root_ref[...] = (jnp.dot(h, wroot_ref[...], preferred_element_type=jnp.float32)
                   + bc_ref[...])


def _tc_encode_l1(x, w1, b1, w2, b2, wr0, wr1, wroot, bc):
  d_in = x.shape[1]
  d_h = w2.shape[1]
  return pl.pallas_call(
      _k1_body,
      grid=(_N // _BN,),
      in_specs=[
          pl.BlockSpec((_BN, d_in), lambda i: (i, 0)),
          _full(w1.shape), _full(b1.shape), _full(w2.shape), _full(b2.shape),
          _full((d_h, _H)), _full((d_h, _H)), _full((d_h, _H)), _full(bc.shape),
      ],
      out_specs=[
          pl.BlockSpec((2, _BN, _H), lambda i: (0, i, 0)),
          pl.BlockSpec((_BN, _H), lambda i: (i, 0)),
      ],
      out_shape=[
          jax.ShapeDtypeStruct((2, _N, _H), jnp.float32),
          jax.ShapeDtypeStruct((_N, _H), jnp.float32),
      ],
  )(x, w1, b1, w2, b2, wr0, wr1, wroot, bc)


def _epilogue(root_ref, a0_ref, a1_ref, scn_ref, we_ref):
  s0 = scn_ref[:, 0:1]
  c0 = scn_ref[:, 1:2]
  s1 = scn_ref[:, 2:3]
  c1 = scn_ref[:, 3:4]
  we = we_ref[...]
  t0 = (a0_ref[...] + s0 * we) / jnp.maximum(c0, 1.0)
  t1 = (a1_ref[...] + s1 * we) / jnp.maximum(c1, 1.0)
  h = root_ref[...] + t0 + t1
  return jnp.where(h > 0.0, h, jnp.exp(jnp.minimum(h, 0.0)) - 1.0)


def _kmid_body(root_ref, a0_ref, a1_ref, scn_ref, we_ref,
               wr0_ref, wr1_ref, wroot_ref, bc_ref, y_ref, rootn_ref):
  h = _epilogue(root_ref, a0_ref, a1_ref, scn_ref, we_ref)
  y_ref[0] = jnp.dot(h, wr0_ref[...], preferred_element_type=jnp.float32)
  y_ref[1] = jnp.dot(h, wr1_ref[...], preferred_element_type=jnp.float32)
  rootn_ref[...] = (jnp.dot(h, wroot_ref[...],
                            preferred_element_type=jnp.float32) + bc_ref[...])


def _tc_mid(root, a0, a1, scn, we, wr0, wr1, wroot, bc):
  return pl.pallas_call(
      _kmid_body,
      grid=(_N // _BN,),
      in_specs=[
          pl.BlockSpec((_BN, _H), lambda i: (i, 0)),
          pl.BlockSpec((_BN, _H), lambda i: (i, 0)),
          pl.BlockSpec((_BN, _H), lambda i: (i, 0)),
          pl.BlockSpec((_BN, 8), lambda i: (i, 0)),
          _full(we.shape),
          _full((_H, _H)), _full((_H, _H)), _full((_H, _H)), _full(bc.shape),
      ],
      out_specs=[
          pl.BlockSpec((2, _BN, _H), lambda i: (0, i, 0)),
          pl.BlockSpec((_BN, _H), lambda i: (i, 0)),
      ],
      out_shape=[
          jax.ShapeDtypeStruct((2, _N, _H), jnp.float32),
          jax.ShapeDtypeStruct((_N, _H), jnp.float32),
      ],
  )(root, a0, a1, scn, we, wr0, wr1, wroot, bc)


def _kfin_body(root_ref, a0_ref, a1_ref, scn_ref, we_ref,
               batch_ref, wl_ref, bl_ref, out_ref, p_acc, c_acc):
  i = pl.program_id(0)

  @pl.when(i == 0)
  def _():
    p_acc[...] = jnp.zeros_like(p_acc)
    c_acc[...] = jnp.zeros_like(c_acc)

  h = _epilogue(root_ref, a0_ref, a1_ref, scn_ref, we_ref)
  bf = batch_ref[...]
  gids = lax.broadcasted_iota(jnp.int32, (_BN, _G), 1).astype(jnp.float32)
  ob = (bf == gids).astype(jnp.float32)
  p_acc[...] += lax.dot_general(ob, h, (((0,), (0,)), ((), ())),
                                preferred_element_type=jnp.float32)
  c_acc[...] += jnp.sum(ob, axis=0)[:, None]

  @pl.when(i == _N // _BN - 1)
  def _():
    pooled = p_acc[...] / jnp.maximum(c_acc[...], 1.0)
    out_ref[...] = (jnp.dot(pooled, wl_ref[...],
                            preferred_element_type=jnp.float32) + bl_ref[...])


def _tc_final(root, a0, a1, scn, we, batchf, wl, bl):
  return pl.pallas_call(
      _kfin_body,
      grid=(_N // _BN,),
      in_specs=[
          pl.BlockSpec((_BN, _H), lambda i: (i, 0)),
          pl.BlockSpec((_BN, _H), lambda i: (i, 0)),
          pl.BlockSpec((_BN, _H), lambda i: (i, 0)),
          pl.BlockSpec((_BN, 8), lambda i: (i, 0)),
          _full(we.shape),
          pl.BlockSpec((_BN, 1), lambda i: (i, 0)),
          _full(wl.shape), _full(bl.shape),
      ],
      out_specs=pl.BlockSpec((_G, _C), lambda i: (0, 0)),
      out_shape=jax.ShapeDtypeStruct((_G, _C), jnp.float32),
      scratch_shapes=[
          pltpu.VMEM((_G, _H), jnp.float32),
          pltpu.VMEM((_G, 1), jnp.float32),
      ],
  )(root, a0, a1, scn, we, batchf, wl, bl)


def kernel(x, edge_index, edge_attr, edge_type, batch,
           W1, b1, W2, b2,
           Wroot1, Wrel1, We1, bc1,
           Wroot2, Wrel2, We2, bc2,
           Wroot3, Wrel3, We3, bc3,
           Wroot4, Wrel4, We4, bc4,
           Wl, bl):
  src = edge_index[0]
  dst = edge_index[1]
  et = edge_type

  pad = _EPAD - _E
  src3 = jnp.pad(src, (0, pad)).reshape(_NSUB, _EPW)
  dst3p = jnp.pad(dst, (0, pad)).reshape(_NSUB, _EPW)
  typ3 = jnp.pad(et, (0, pad), constant_values=2).reshape(_NSUB, _EPW)

  srcb, dstb, nchb = _sc_compact(src3, dst3p, typ3)

  dst0 = jnp.where(et == 0, dst, _STRASH)
  dst1 = jnp.where(et == 1, dst, _STRASH)
  dst3 = jnp.stack([
      jnp.pad(dst0, (0, pad), constant_values=_STRASH),
      jnp.pad(dst1, (0, pad), constant_values=_STRASH),
  ]).reshape(2, _NSUB, _NCH, _CH)
  av = jnp.pad(jnp.concatenate(
      [edge_attr.astype(jnp.float32),
       jnp.ones((_E, 1), jnp.float32)], axis=1), ((0, pad), (0, 14)))
  av3 = av.reshape(_NSUB, _NCH, _CH, 16)

  stats = _sc_edge_stats(av3, dst3)
  scn = jnp.concatenate([
      stats[0, :_N, 0:2], stats[1, :_N, 0:2],
      jnp.zeros((_N, 4), jnp.float32)], axis=1)

  x = x.astype(jnp.float32)
  y, root = _tc_encode_l1(x, W1, b1.reshape(1, -1), W2, b2.reshape(1, -1),
                          Wrel1[0], Wrel1[1], Wroot1, bc1.reshape(1, -1))

  def aggs(y):
    out = _sc_scatter(y.reshape(2 * _N, _H), srcb, dstb, nchb)
    a = out.reshape(2, _NQ * _QR, 128)
    return a[0, :_N], a[1, :_N]

  for Wroot, Wrel, We, bc in ((Wroot2, Wrel2, We1, bc2),
                              (Wroot3, Wrel3, We2, bc3),
                              (Wroot4, Wrel4, We3, bc4)):
    a0, a1 = aggs(y)
    y, root = _tc_mid(root, a0, a1, scn, We.reshape(1, -1),
                      Wrel[0], Wrel[1], Wroot, bc.reshape(1, -1))

  a0, a1 = aggs(y)
  batchf = batch.astype(jnp.float32).reshape(_N, 1)
  return _tc_final(root, a0, a1, scn, We4.reshape(1, -1),
                   batchf, Wl, bl.reshape(1, -1))


# speedup vs baseline: 2.6110x; 1.0993x over previous
import functools

import jax
import jax.numpy as jnp
from jax import lax
from jax.experimental import pallas as pl
from jax.experimental.pallas import tpu as pltpu
from jax.experimental.pallas import tpu_sc as plsc

_N = 10000
_E = 320000
_H = 128
_G = 64
_C = 10

_NSUB = 16
_CH = 128
_EPW = 20480
_NCH = _EPW // _CH
_EPAD = _NSUB * _EPW
_NQ = 5
_QR = 2048
_ACCQ = 2176
_QTRASH = 2100
_QSTR = _QR // _NSUB
_CB = 20736

_SACC = 10240
_STRASH = _N
_SSTR = _SACC // _NSUB
_ZR = 64


def _zero_vmem(ref, rows, width):
  @pl.loop(0, rows)
  def _(r):
    @pl.loop(0, width // 16)
    def _(k):
      ref[r, pl.ds(k * 16, 16)] = jnp.zeros((16,), jnp.float32)


@functools.lru_cache(maxsize=None)
def _sc_kernels():
  mesh = plsc.VectorSubcoreMesh(core_axis_name="c", subcore_axis_name="s",
                                num_cores=2, num_subcores=_NSUB)
  params = pltpu.CompilerParams(use_tc_tiling_on_sc=False,
                                needs_layout_passes=False)

  @functools.partial(
      pl.kernel,
      out_type=[
          jax.ShapeDtypeStruct((2, _NQ, _NSUB, _NCH, _CH), jnp.int32),
          jax.ShapeDtypeStruct((2, _NQ, _NSUB, _NCH, _CH), jnp.int32),
          jax.ShapeDtypeStruct((2, _NQ, _NSUB, _NCH, _CH), jnp.float32),
          jax.ShapeDtypeStruct((2, _NQ, _NSUB, 16), jnp.int32),
      ],
      mesh=mesh,
      compiler_params=params,
      scratch_types=[
          pltpu.VMEM((_EPW,), jnp.int32),
          pltpu.VMEM((_EPW,), jnp.int32),
          pltpu.VMEM((_EPW,), jnp.float32),
          pltpu.VMEM((_NCH + 3, _CH), jnp.int32),
          pltpu.VMEM((_NCH + 3, _CH), jnp.int32),
          pltpu.VMEM((_NCH + 3, _CH), jnp.float32),
          pltpu.VMEM((16,), jnp.int32),
      ],
  )
  def sc_compact(src_hbm, dst_hbm, atr_hbm,
                 srcb_hbm, dstb_hbm, atrb_hbm, nch_hbm,
                 src_sl, dst_sl, atr_sl, sbuf, dbuf, abuf, cnt_v):
    c = lax.axis_index("c")
    s = lax.axis_index("s")
    pltpu.sync_copy(src_hbm.at[s], src_sl)
    pltpu.sync_copy(dst_hbm.at[s], dst_sl)
    pltpu.sync_copy(atr_hbm.at[s], atr_sl)
    lanes = lax.iota(jnp.int32, 16)
    srow = c * _N

    for q in range(_NQ):
      @plsc.parallel_loop(0, _EPW // 16, carry=jnp.int32(0))
      def o(i, o_in):
        li = 16 * i + lanes
        sc_ = plsc.load_gather(src_sl, [li])
        dv = plsc.load_gather(dst_sl, [li])
        tv = sc_ >> 14
        sv = sc_ & 16383
        m = (tv == c) & (dv >= _QR * q) & (dv < _QR * (q + 1))
        mi = m.astype(jnp.int32)
        idx = o_in + plsc.cumsum(mi) - 1
        av = plsc.load_gather(atr_sl, [li])
        plsc.store_scatter(sbuf, [idx >> 7, idx & 127], sv + srow, mask=m)
        plsc.store_scatter(dbuf, [idx >> 7, idx & 127], dv - _QR * q, mask=m)
        plsc.store_scatter(abuf, [idx >> 7, idx & 127], av, mask=m)
        return o_in + jnp.sum(mi)

      ones = jnp.full((16,), True)
      for t in range(_CH // 16 + 1):
        pidx = o + lanes + 16 * t
        plsc.store_scatter(sbuf, [pidx >> 7, pidx & 127],
                           jnp.full((16,), 0, jnp.int32) + srow, mask=ones)
        plsc.store_scatter(dbuf, [pidx >> 7, pidx & 127],
                           jnp.full((16,), _QTRASH, jnp.int32), mask=ones)
        plsc.store_scatter(abuf, [pidx >> 7, pidx & 127],
                           jnp.zeros((16,), jnp.float32), mask=ones)
      nch = (o + _CH - 1) // _CH

      pltpu.sync_copy(sbuf.at[pl.ds(0, _NCH)], srcb_hbm.at[c, q, s])
      pltpu.sync_copy(dbuf.at[pl.ds(0, _NCH)], dstb_hbm.at[c, q, s])
      pltpu.sync_copy(abuf.at[pl.ds(0, _NCH)], atrb_hbm.at[c, q, s])
      cnt_v[...] = jnp.zeros((16,), jnp.int32) + nch
      pltpu.sync_copy(cnt_v, nch_hbm.at[c, q, s])

  @functools.partial(
      pl.kernel,
      out_type=jax.ShapeDtypeStruct((2, _NQ, _QR, 128), jnp.float32),
      mesh=mesh,
      compiler_params=params,
      scratch_types=[
          pltpu.VMEM((_NCH, _CH), jnp.int32),
          pltpu.VMEM((_NCH, _CH), jnp.int32),
          pltpu.VMEM((2, _CH, 128), jnp.float32),
          pltpu.VMEM((_QSTR // 2, 128), jnp.float32),
          pltpu.VMEM((16,), jnp.int32),
          pltpu.VMEM_SHARED((_ACCQ, 128), jnp.float32),
          [pltpu.SemaphoreType.DMA] * 2,
          [pltpu.SemaphoreType.DMA] * 2,
      ],
  )
  def sc_scatter(y_hbm, srcb_hbm, dstb_hbm, nch_hbm, out_hbm,
                 src_v, dst_v, rows_v, z_v, cnt_v, acc_sh, gsems, ssems):
    c = lax.axis_index("c")
    s = lax.axis_index("s")
    _zero_vmem(z_v, _QSTR // 2, 128)
    base = s * _QSTR

    for q in range(_NQ):
      pltpu.sync_copy(nch_hbm.at[c, q, s], cnt_v)
      nch = jnp.max(cnt_v[...])
      for p in range(4):
        @pl.when(p * 40 < nch)
        def _():
          pltpu.sync_copy(srcb_hbm.at[c, q, s, pl.ds(p * 40, 40)],
                          src_v.at[pl.ds(p * 40, 40)])
          pltpu.sync_copy(dstb_hbm.at[c, q, s, pl.ds(p * 40, 40)],
                          dst_v.at[pl.ds(p * 40, 40)])
      for hh in range(2):
        pltpu.sync_copy(z_v, acc_sh.at[pl.ds(base + hh * (_QSTR // 2),
                                             _QSTR // 2)])
      plsc.subcore_barrier()

      for b in range(2):
        @pl.when(b < nch)
        def _():
          pltpu.async_copy(y_hbm.at[src_v.at[b]], rows_v.at[b], gsems[b])

      @pl.loop(0, (nch + 1) // 2)
      def _(jr):
        j = 2 * jr
        for b in range(2):
          @pl.when(j + b < nch)
          def _():
            pltpu.make_async_copy(y_hbm.at[src_v.at[0]], rows_v.at[b],
                                  gsems[b]).wait()
            pltpu.async_copy(rows_v.at[b], acc_sh.at[dst_v.at[j + b]],
                             ssems[b], add=True)
        for b in range(2):
          @pl.when(j + b + 2 < nch)
          def _():
            pltpu.make_async_copy(y_hbm.at[src_v.at[0]], rows_v.at[b],
                                  ssems[b]).wait()
            pltpu.async_copy(y_hbm.at[src_v.at[j + b + 2]], rows_v.at[b],
                             gsems[b])

      for b in range(2):
        @pl.when(b < jnp.minimum(nch, 2))
        def _():
          pltpu.make_async_copy(y_hbm.at[src_v.at[0]], rows_v.at[b],
                                ssems[b]).wait()

      plsc.subcore_barrier()
      pltpu.sync_copy(acc_sh.at[pl.ds(base, _QSTR)],
                      out_hbm.at[c, q, pl.ds(base, _QSTR)])

  @functools.partial(
      pl.kernel,
      out_type=jax.ShapeDtypeStruct((2, _NQ, _QR, 16), jnp.float32),
      mesh=mesh,
      compiler_params=params,
      scratch_types=[
          pltpu.VMEM((_NCH, _CH), jnp.int32),
          pltpu.VMEM((_NCH, _CH), jnp.float32),
          pltpu.VMEM((_CH, 16), jnp.float32),
          pltpu.VMEM((_QSTR // 2, 16), jnp.float32),
          pltpu.VMEM((16,), jnp.int32),
          pltpu.VMEM_SHARED((_ACCQ, 16), jnp.float32),
      ],
  )
  def sc_stats2(atrb_hbm, dstb_hbm, nch_hbm, out_hbm,
                dst_v, atr_v, rowb, z_v, cnt_v, acc_sh):
    c = lax.axis_index("c")
    s = lax.axis_index("s")
    lanes = lax.iota(jnp.int32, 16)
    _zero_vmem(z_v, _QSTR // 2, 16)
    _zero_vmem(rowb, _CH, 16)
    for k in range(_CH // 16):
      plsc.store_scatter(rowb, [k * 16 + lanes, jnp.full((16,), 1, jnp.int32)],
                         jnp.full((16,), 1.0, jnp.float32))
    base = s * _QSTR

    for q in range(_NQ):
      pltpu.sync_copy(nch_hbm.at[c, q, s], cnt_v)
      nch = jnp.max(cnt_v[...])
      for p in range(4):
        @pl.when(p * 40 < nch)
        def _():
          pltpu.sync_copy(atrb_hbm.at[c, q, s, pl.ds(p * 40, 40)],
                          atr_v.at[pl.ds(p * 40, 40)])
          pltpu.sync_copy(dstb_hbm.at[c, q, s, pl.ds(p * 40, 40)],
                          dst_v.at[pl.ds(p * 40, 40)])
      for hh in range(2):
        pltpu.sync_copy(z_v, acc_sh.at[pl.ds(base + hh * (_QSTR // 2),
                                             _QSTR // 2)])
      plsc.subcore_barrier()

      @pl.loop(0, nch)
      def _(j):
        for k in range(_CH // 16):
          av = plsc.load_gather(atr_v, [jnp.full((16,), 0, jnp.int32) + j,
                                        k * 16 + lanes])
          plsc.store_scatter(rowb, [k * 16 + lanes,
                                    jnp.zeros((16,), jnp.int32)], av)
        pltpu.sync_copy(rowb, acc_sh.at[dst_v.at[j]], add=True)

      plsc.subcore_barrier()
      pltpu.sync_copy(acc_sh.at[pl.ds(base, _QSTR)],
                      out_hbm.at[c, q, pl.ds(base, _QSTR)])

  return sc_compact, sc_scatter, sc_stats2


def _sc_compact(src3, dst3p, atr3):
  return _sc_kernels()[0](src3, dst3p, atr3)


def _sc_scatter(y, srcb, dstb, nchb):
  return _sc_kernels()[1](y, srcb, dstb, nchb)


def _sc_stats2(atrb, dstb, nchb):
  return _sc_kernels()[2](atrb, dstb, nchb)


_BN = 1000


def _full(shape):
  return pl.BlockSpec(shape, lambda i: (0,) * len(shape))


def _k1_body(x_ref, w1_ref, b1_ref, w2_ref, b2_ref,
             wr0_ref, wr1_ref, wroot_ref, bc_ref, y_ref, root_ref):
  x = x_ref[...]
  t = jnp.dot(x, w1_ref[...], preferred_element_type=jnp.float32) + b1_ref[...]
  h = jnp.dot(t, w2_ref[...], preferred_element_type=jnp.float32) + b2_ref[...]
  y_ref[0] = jnp.dot(h, wr0_ref[...], preferred_element_type=jnp.float32)
  y_ref[1] = jnp.dot(h, wr1_ref[...], preferred_element_type=jnp.float32)
  root_ref[...] = (jnp.dot(h, wroot_ref[...], preferred_element_type=jnp.float32)
                   + bc_ref[...])


def _tc_encode_l1(x, w1, b1, w2, b2, wr0, wr1, wroot, bc):
  d_in = x.shape[1]
  d_h = w2.shape[1]
  return pl.pallas_call(
      _k1_body,
      grid=(_N // _BN,),
      in_specs=[
          pl.BlockSpec((_BN, d_in), lambda i: (i, 0)),
          _full(w1.shape), _full(b1.shape), _full(w2.shape), _full(b2.shape),
          _full((d_h, _H)), _full((d_h, _H)), _full((d_h, _H)), _full(bc.shape),
      ],
      out_specs=[
          pl.BlockSpec((2, _BN, _H), lambda i: (0, i, 0)),
          pl.BlockSpec((_BN, _H), lambda i: (i, 0)),
      ],
      out_shape=[
          jax.ShapeDtypeStruct((2, _N, _H), jnp.float32),
          jax.ShapeDtypeStruct((_N, _H), jnp.float32),
      ],
  )(x, w1, b1, w2, b2, wr0, wr1, wroot, bc)


def _epilogue(root_ref, a0_ref, a1_ref, scn_ref, we_ref):
  s0 = scn_ref[:, 0:1]
  c0 = scn_ref[:, 1:2]
  s1 = scn_ref[:, 2:3]
  c1 = scn_ref[:, 3:4]
  we = we_ref[...]
  t0 = (a0_ref[...] + s0 * we) / jnp.maximum(c0, 1.0)
  t1 = (a1_ref[...] + s1 * we) / jnp.maximum(c1, 1.0)
  h = root_ref[...] + t0 + t1
  return jnp.where(h > 0.0, h, jnp.exp(jnp.minimum(h, 0.0)) - 1.0)


def _kmid_body(root_ref, a0_ref, a1_ref, scn_ref, we_ref,
               wr0_ref, wr1_ref, wroot_ref, bc_ref, y_ref, rootn_ref):
  h = _epilogue(root_ref, a0_ref, a1_ref, scn_ref, we_ref)
  y_ref[0] = jnp.dot(h, wr0_ref[...], preferred_element_type=jnp.float32)
  y_ref[1] = jnp.dot(h, wr1_ref[...], preferred_element_type=jnp.float32)
  rootn_ref[...] = (jnp.dot(h, wroot_ref[...],
                            preferred_element_type=jnp.float32) + bc_ref[...])


def _tc_mid(root, a0, a1, scn, we, wr0, wr1, wroot, bc):
  return pl.pallas_call(
      _kmid_body,
      grid=(_N // _BN,),
      in_specs=[
          pl.BlockSpec((_BN, _H), lambda i: (i, 0)),
          pl.BlockSpec((_BN, _H), lambda i: (i, 0)),
          pl.BlockSpec((_BN, _H), lambda i: (i, 0)),
          pl.BlockSpec((_BN, 8), lambda i: (i, 0)),
          _full(we.shape),
          _full((_H, _H)), _full((_H, _H)), _full((_H, _H)), _full(bc.shape),
      ],
      out_specs=[
          pl.BlockSpec((2, _BN, _H), lambda i: (0, i, 0)),
          pl.BlockSpec((_BN, _H), lambda i: (i, 0)),
      ],
      out_shape=[
          jax.ShapeDtypeStruct((2, _N, _H), jnp.float32),
          jax.ShapeDtypeStruct((_N, _H), jnp.float32),
      ],
  )(root, a0, a1, scn, we, wr0, wr1, wroot, bc)


def _kfin_body(root_ref, a0_ref, a1_ref, scn_ref, we_ref,
               batch_ref, wl_ref, bl_ref, out_ref, p_acc, c_acc):
  i = pl.program_id(0)

  @pl.when(i == 0)
  def _():
    p_acc[...] = jnp.zeros_like(p_acc)
    c_acc[...] = jnp.zeros_like(c_acc)

  h = _epilogue(root_ref, a0_ref, a1_ref, scn_ref, we_ref)
  bf = batch_ref[...]
  gids = lax.broadcasted_iota(jnp.int32, (_BN, _G), 1).astype(jnp.float32)
  ob = (bf == gids).astype(jnp.float32)
  p_acc[...] += lax.dot_general(ob, h, (((0,), (0,)), ((), ())),
                                preferred_element_type=jnp.float32)
  c_acc[...] += jnp.sum(ob, axis=0)[:, None]

  @pl.when(i == _N // _BN - 1)
  def _():
    pooled = p_acc[...] / jnp.maximum(c_acc[...], 1.0)
    out_ref[...] = (jnp.dot(pooled, wl_ref[...],
                            preferred_element_type=jnp.float32) + bl_ref[...])


def _tc_final(root, a0, a1, scn, we, batchf, wl, bl):
  return pl.pallas_call(
      _kfin_body,
      grid=(_N // _BN,),
      in_specs=[
          pl.BlockSpec((_BN, _H), lambda i: (i, 0)),
          pl.BlockSpec((_BN, _H), lambda i: (i, 0)),
          pl.BlockSpec((_BN, _H), lambda i: (i, 0)),
          pl.BlockSpec((_BN, 8), lambda i: (i, 0)),
          _full(we.shape),
          pl.BlockSpec((_BN, 1), lambda i: (i, 0)),
          _full(wl.shape), _full(bl.shape),
      ],
      out_specs=pl.BlockSpec((_G, _C), lambda i: (0, 0)),
      out_shape=jax.ShapeDtypeStruct((_G, _C), jnp.float32),
      scratch_shapes=[
          pltpu.VMEM((_G, _H), jnp.float32),
          pltpu.VMEM((_G, 1), jnp.float32),
      ],
  )(root, a0, a1, scn, we, batchf, wl, bl)


def kernel(x, edge_index, edge_attr, edge_type, batch,
           W1, b1, W2, b2,
           Wroot1, Wrel1, We1, bc1,
           Wroot2, Wrel2, We2, bc2,
           Wroot3, Wrel3, We3, bc3,
           Wroot4, Wrel4, We4, bc4,
           Wl, bl):
  src = edge_index[0]
  dst = edge_index[1]
  et = edge_type

  pad = _EPAD - _E
  src3 = jnp.pad(src + (et << 14), (0, pad),
                 constant_values=(2 << 14)).reshape(_NSUB, _EPW)
  dst3p = jnp.pad(dst, (0, pad)).reshape(_NSUB, _EPW)

  atr3 = jnp.pad(edge_attr[:, 0].astype(jnp.float32),
                 (0, pad)).reshape(_NSUB, _EPW)
  srcb, dstb, atrb, nchb = _sc_compact(src3, dst3p, atr3)

  stats = _sc_stats2(atrb, dstb, nchb).reshape(2, _NQ * _QR, 16)
  scn = jnp.concatenate([
      stats[0, :_N, 0:2], stats[1, :_N, 0:2],
      jnp.zeros((_N, 4), jnp.float32)], axis=1)

  x = x.astype(jnp.float32)
  y, root = _tc_encode_l1(x, W1, b1.reshape(1, -1), W2, b2.reshape(1, -1),
                          Wrel1[0], Wrel1[1], Wroot1, bc1.reshape(1, -1))

  def aggs(y):
    out = _sc_scatter(y.reshape(2 * _N, _H), srcb, dstb, nchb)
    a = out.reshape(2, _NQ * _QR, 128)
    return a[0, :_N], a[1, :_N]

  for Wroot, Wrel, We, bc in ((Wroot2, Wrel2, We1, bc2),
                              (Wroot3, Wrel3, We2, bc3),
                              (Wroot4, Wrel4, We3, bc4)):
    a0, a1 = aggs(y)
    y, root = _tc_mid(root, a0, a1, scn, We.reshape(1, -1),
                      Wrel[0], Wrel[1], Wroot, bc.reshape(1, -1))

  a0, a1 = aggs(y)
  batchf = batch.astype(jnp.float32).reshape(_N, 1)
  return _tc_final(root, a0, a1, scn, We4.reshape(1, -1),
                   batchf, Wl, bl.reshape(1, -1))


# speedup vs baseline: 2.7595x; 1.0569x over previous
import functools

import jax
import jax.numpy as jnp
from jax import lax
from jax.experimental import pallas as pl
from jax.experimental.pallas import tpu as pltpu
from jax.experimental.pallas import tpu_sc as plsc

_N = 10000
_E = 320000
_H = 128
_G = 64
_C = 10

_NSUB = 16
_CH = 128
_EPW = 20480
_NCH = _EPW // _CH
_EPAD = _NSUB * _EPW
_NQ = 5
_QR = 2048
_ACCQ = 2176
_QTRASH = 2100
_QSTR = _QR // _NSUB
_CB = 20736

_SACC = 10240
_STRASH = _N
_SSTR = _SACC // _NSUB
_ZR = 64


def _zero_vmem(ref, rows, width):
  @pl.loop(0, rows)
  def _(r):
    @pl.loop(0, width // 16)
    def _(k):
      ref[r, pl.ds(k * 16, 16)] = jnp.zeros((16,), jnp.float32)


@functools.lru_cache(maxsize=None)
def _sc_kernels():
  mesh = plsc.VectorSubcoreMesh(core_axis_name="c", subcore_axis_name="s",
                                num_cores=2, num_subcores=_NSUB)
  params = pltpu.CompilerParams(use_tc_tiling_on_sc=False,
                                needs_layout_passes=False)

  @functools.partial(
      pl.kernel,
      out_type=[
          jax.ShapeDtypeStruct((2, _NQ, _NSUB, _NCH, _CH), jnp.int32),
          jax.ShapeDtypeStruct((2, _NQ, _NSUB, _NCH, _CH), jnp.int32),
          jax.ShapeDtypeStruct((2, _NQ, _NSUB, _NCH, _CH), jnp.float32),
          jax.ShapeDtypeStruct((2, _NQ, _NSUB, 16), jnp.int32),
      ],
      mesh=mesh,
      compiler_params=params,
      scratch_types=[
          pltpu.VMEM((_EPW,), jnp.int32),
          pltpu.VMEM((_EPW,), jnp.int32),
          pltpu.VMEM((_EPW,), jnp.float32),
          pltpu.VMEM((_NCH + 3, _CH), jnp.int32),
          pltpu.VMEM((_NCH + 3, _CH), jnp.int32),
          pltpu.VMEM((_NCH + 3, _CH), jnp.float32),
          pltpu.VMEM((16,), jnp.int32),
      ],
  )
  def sc_compact(src_hbm, dst_hbm, atr_hbm,
                 srcb_hbm, dstb_hbm, atrb_hbm, nch_hbm,
                 src_sl, dst_sl, atr_sl, sbuf, dbuf, abuf, cnt_v):
    c = lax.axis_index("c")
    s = lax.axis_index("s")
    pltpu.sync_copy(src_hbm.at[s], src_sl)
    pltpu.sync_copy(dst_hbm.at[s], dst_sl)
    pltpu.sync_copy(atr_hbm.at[s], atr_sl)
    lanes = lax.iota(jnp.int32, 16)
    srow = c * _N

    for q in range(_NQ):
      @plsc.parallel_loop(0, _EPW // 16, carry=jnp.int32(0))
      def o(i, o_in):
        li = 16 * i + lanes
        sc_ = plsc.load_gather(src_sl, [li])
        dv = plsc.load_gather(dst_sl, [li])
        tv = sc_ >> 14
        sv = sc_ & 16383
        m = (tv == c) & (dv >= _QR * q) & (dv < _QR * (q + 1))
        mi = m.astype(jnp.int32)
        idx = o_in + plsc.cumsum(mi) - 1
        av = plsc.load_gather(atr_sl, [li])
        plsc.store_scatter(sbuf, [idx >> 7, idx & 127], sv + srow, mask=m)
        plsc.store_scatter(dbuf, [idx >> 7, idx & 127], dv - _QR * q, mask=m)
        plsc.store_scatter(abuf, [idx >> 7, idx & 127], av, mask=m)
        return o_in + jnp.sum(mi)

      ones = jnp.full((16,), True)
      for t in range(_CH // 16 + 1):
        pidx = o + lanes + 16 * t
        plsc.store_scatter(sbuf, [pidx >> 7, pidx & 127],
                           jnp.full((16,), 0, jnp.int32) + srow, mask=ones)
        plsc.store_scatter(dbuf, [pidx >> 7, pidx & 127],
                           jnp.full((16,), _QTRASH, jnp.int32), mask=ones)
        plsc.store_scatter(abuf, [pidx >> 7, pidx & 127],
                           jnp.zeros((16,), jnp.float32), mask=ones)
      nch = (o + _CH - 1) // _CH

      pltpu.sync_copy(sbuf.at[pl.ds(0, _NCH)], srcb_hbm.at[c, q, s])
      pltpu.sync_copy(dbuf.at[pl.ds(0, _NCH)], dstb_hbm.at[c, q, s])
      pltpu.sync_copy(abuf.at[pl.ds(0, _NCH)], atrb_hbm.at[c, q, s])
      cnt_v[...] = jnp.zeros((16,), jnp.int32) + nch
      pltpu.sync_copy(cnt_v, nch_hbm.at[c, q, s])

  @functools.partial(
      pl.kernel,
      out_type=jax.ShapeDtypeStruct((2, _NQ, _QR, 128), jnp.float32),
      mesh=mesh,
      compiler_params=params,
      scratch_types=[
          pltpu.VMEM((_NCH, _CH), jnp.int32),
          pltpu.VMEM((_NCH, _CH), jnp.int32),
          pltpu.VMEM((3, _CH, 128), jnp.float32),
          pltpu.VMEM((_QSTR // 2, 128), jnp.float32),
          pltpu.VMEM((16,), jnp.int32),
          pltpu.VMEM_SHARED((_ACCQ, 128), jnp.float32),
          [pltpu.SemaphoreType.DMA] * 3,
          [pltpu.SemaphoreType.DMA] * 3,
      ],
  )
  def sc_scatter(y_hbm, srcb_hbm, dstb_hbm, nch_hbm, out_hbm,
                 src_v, dst_v, rows_v, z_v, cnt_v, acc_sh, gsems, ssems):
    c = lax.axis_index("c")
    s = lax.axis_index("s")
    _zero_vmem(z_v, _QSTR // 2, 128)
    base = s * _QSTR

    for q in range(_NQ):
      pltpu.sync_copy(nch_hbm.at[c, q, s], cnt_v)
      nch = jnp.max(cnt_v[...])
      for p in range(4):
        @pl.when(p * 40 < nch)
        def _():
          pltpu.sync_copy(srcb_hbm.at[c, q, s, pl.ds(p * 40, 40)],
                          src_v.at[pl.ds(p * 40, 40)])
          pltpu.sync_copy(dstb_hbm.at[c, q, s, pl.ds(p * 40, 40)],
                          dst_v.at[pl.ds(p * 40, 40)])
      for hh in range(2):
        pltpu.sync_copy(z_v, acc_sh.at[pl.ds(base + hh * (_QSTR // 2),
                                             _QSTR // 2)])
      plsc.subcore_barrier()

      for b in range(3):
        @pl.when(b < nch)
        def _():
          pltpu.async_copy(y_hbm.at[src_v.at[b]], rows_v.at[b], gsems[b])

      @pl.loop(0, (nch + 2) // 3)
      def _(jr):
        j = 3 * jr
        for b in range(3):
          @pl.when(j + b < nch)
          def _():
            pltpu.make_async_copy(y_hbm.at[src_v.at[0]], rows_v.at[b],
                                  gsems[b]).wait()
            pltpu.async_copy(rows_v.at[b], acc_sh.at[dst_v.at[j + b]],
                             ssems[b], add=True)
        for b in range(3):
          @pl.when(j + b + 3 < nch)
          def _():
            pltpu.make_async_copy(y_hbm.at[src_v.at[0]], rows_v.at[b],
                                  ssems[b]).wait()
            pltpu.async_copy(y_hbm.at[src_v.at[j + b + 3]], rows_v.at[b],
                             gsems[b])

      for b in range(3):
        @pl.when(b < jnp.minimum(nch, 3))
        def _():
          pltpu.make_async_copy(y_hbm.at[src_v.at[0]], rows_v.at[b],
                                ssems[b]).wait()

      plsc.subcore_barrier()
      pltpu.sync_copy(acc_sh.at[pl.ds(base, _QSTR)],
                      out_hbm.at[c, q, pl.ds(base, _QSTR)])

  @functools.partial(
      pl.kernel,
      out_type=jax.ShapeDtypeStruct((2, _NQ, _QR, 16), jnp.float32),
      mesh=mesh,
      compiler_params=params,
      scratch_types=[
          pltpu.VMEM((_NCH, _CH), jnp.int32),
          pltpu.VMEM((_NCH, _CH), jnp.float32),
          pltpu.VMEM((_CH, 16), jnp.float32),
          pltpu.VMEM((_QSTR // 2, 16), jnp.float32),
          pltpu.VMEM((16,), jnp.int32),
          pltpu.VMEM_SHARED((_ACCQ, 16), jnp.float32),
      ],
  )
  def sc_stats2(atrb_hbm, dstb_hbm, nch_hbm, out_hbm,
                dst_v, atr_v, rowb, z_v, cnt_v, acc_sh):
    c = lax.axis_index("c")
    s = lax.axis_index("s")
    lanes = lax.iota(jnp.int32, 16)
    _zero_vmem(z_v, _QSTR // 2, 16)
    _zero_vmem(rowb, _CH, 16)
    for k in range(_CH // 16):
      plsc.store_scatter(rowb, [k * 16 + lanes, jnp.full((16,), 1, jnp.int32)],
                         jnp.full((16,), 1.0, jnp.float32))
    base = s * _QSTR

    for q in range(_NQ):
      pltpu.sync_copy(nch_hbm.at[c, q, s], cnt_v)
      nch = jnp.max(cnt_v[...])
      for p in range(4):
        @pl.when(p * 40 < nch)
        def _():
          pltpu.sync_copy(atrb_hbm.at[c, q, s, pl.ds(p * 40, 40)],
                          atr_v.at[pl.ds(p * 40, 40)])
          pltpu.sync_copy(dstb_hbm.at[c, q, s, pl.ds(p * 40, 40)],
                          dst_v.at[pl.ds(p * 40, 40)])
      for hh in range(2):
        pltpu.sync_copy(z_v, acc_sh.at[pl.ds(base + hh * (_QSTR // 2),
                                             _QSTR // 2)])
      plsc.subcore_barrier()

      @pl.loop(0, nch)
      def _(j):
        for k in range(_CH // 16):
          av = plsc.load_gather(atr_v, [jnp.full((16,), 0, jnp.int32) + j,
                                        k * 16 + lanes])
          plsc.store_scatter(rowb, [k * 16 + lanes,
                                    jnp.zeros((16,), jnp.int32)], av)
        pltpu.sync_copy(rowb, acc_sh.at[dst_v.at[j]], add=True)

      plsc.subcore_barrier()
      pltpu.sync_copy(acc_sh.at[pl.ds(base, _QSTR)],
                      out_hbm.at[c, q, pl.ds(base, _QSTR)])

  return sc_compact, sc_scatter, sc_stats2


def _sc_compact(src3, dst3p, atr3):
  return _sc_kernels()[0](src3, dst3p, atr3)


def _sc_scatter(y, srcb, dstb, nchb):
  return _sc_kernels()[1](y, srcb, dstb, nchb)


def _sc_stats2(atrb, dstb, nchb):
  return _sc_kernels()[2](atrb, dstb, nchb)


_BN = 1000


def _full(shape):
  return pl.BlockSpec(shape, lambda i: (0,) * len(shape))


def _k1_body(x_ref, w1_ref, b1_ref, w2_ref, b2_ref,
             wr0_ref, wr1_ref, wroot_ref, bc_ref, y_ref, root_ref):
  x = x_ref[...]
  t = jnp.dot(x, w1_ref[...], preferred_element_type=jnp.float32) + b1_ref[...]
  h = jnp.dot(t, w2_ref[...], preferred_element_type=jnp.float32) + b2_ref[...]
  y_ref[0] = jnp.dot(h, wr0_ref[...], preferred_element_type=jnp.float32)
  y_ref[1] = jnp.dot(h, wr1_ref[...], preferred_element_type=jnp.float32)
  root_ref[...] = (jnp.dot(h, wroot_ref[...], preferred_element_type=jnp.float32)
                   + bc_ref[...])


def _tc_encode_l1(x, w1, b1, w2, b2, wr0, wr1, wroot, bc):
  d_in = x.shape[1]
  d_h = w2.shape[1]
  return pl.pallas_call(
      _k1_body,
      grid=(_N // _BN,),
      in_specs=[
          pl.BlockSpec((_BN, d_in), lambda i: (i, 0)),
          _full(w1.shape), _full(b1.shape), _full(w2.shape), _full(b2.shape),
          _full((d_h, _H)), _full((d_h, _H)), _full((d_h, _H)), _full(bc.shape),
      ],
      out_specs=[
          pl.BlockSpec((2, _BN, _H), lambda i: (0, i, 0)),
          pl.BlockSpec((_BN, _H), lambda i: (i, 0)),
      ],
      out_shape=[
          jax.ShapeDtypeStruct((2, _N, _H), jnp.float32),
          jax.ShapeDtypeStruct((_N, _H), jnp.float32),
      ],
  )(x, w1, b1, w2, b2, wr0, wr1, wroot, bc)


def _epilogue(root_ref, a0_ref, a1_ref, scn_ref, we_ref):
  s0 = scn_ref[:, 0:1]
  c0 = scn_ref[:, 1:2]
  s1 = scn_ref[:, 2:3]
  c1 = scn_ref[:, 3:4]
  we = we_ref[...]
  t0 = (a0_ref[...] + s0 * we) / jnp.maximum(c0, 1.0)
  t1 = (a1_ref[...] + s1 * we) / jnp.maximum(c1, 1.0)
  h = root_ref[...] + t0 + t1
  return jnp.where(h > 0.0, h, jnp.exp(jnp.minimum(h, 0.0)) - 1.0)


def _kmid_body(root_ref, a0_ref, a1_ref, scn_ref, we_ref,
               wr0_ref, wr1_ref, wroot_ref, bc_ref, y_ref, rootn_ref):
  h = _epilogue(root_ref, a0_ref, a1_ref, scn_ref, we_ref)
  y_ref[0] = jnp.dot(h, wr0_ref[...], preferred_element_type=jnp.float32)
  y_ref[1] = jnp.dot(h, wr1_ref[...], preferred_element_type=jnp.float32)
  rootn_ref[...] = (jnp.dot(h, wroot_ref[...],
                            preferred_element_type=jnp.float32) + bc_ref[...])


def _tc_mid(root, a0, a1, scn, we, wr0, wr1, wroot, bc):
  return pl.pallas_call(
      _kmid_body,
      grid=(_N // _BN,),
      in_specs=[
          pl.BlockSpec((_BN, _H), lambda i: (i, 0)),
          pl.BlockSpec((_BN, _H), lambda i: (i, 0)),
          pl.BlockSpec((_BN, _H), lambda i: (i, 0)),
          pl.BlockSpec((_BN, 8), lambda i: (i, 0)),
          _full(we.shape),
          _full((_H, _H)), _full((_H, _H)), _full((_H, _H)), _full(bc.shape),
      ],
      out_specs=[
          pl.BlockSpec((2, _BN, _H), lambda i: (0, i, 0)),
          pl.BlockSpec((_BN, _H), lambda i: (i, 0)),
      ],
      out_shape=[
          jax.ShapeDtypeStruct((2, _N, _H), jnp.float32),
          jax.ShapeDtypeStruct((_N, _H), jnp.float32),
      ],
  )(root, a0, a1, scn, we, wr0, wr1, wroot, bc)


def _kfin_body(root_ref, a0_ref, a1_ref, scn_ref, we_ref,
               batch_ref, wl_ref, bl_ref, out_ref, p_acc, c_acc):
  i = pl.program_id(0)

  @pl.when(i == 0)
  def _():
    p_acc[...] = jnp.zeros_like(p_acc)
    c_acc[...] = jnp.zeros_like(c_acc)

  h = _epilogue(root_ref, a0_ref, a1_ref, scn_ref, we_ref)
  bf = batch_ref[...]
  gids = lax.broadcasted_iota(jnp.int32, (_BN, _G), 1).astype(jnp.float32)
  ob = (bf == gids).astype(jnp.float32)
  p_acc[...] += lax.dot_general(ob, h, (((0,), (0,)), ((), ())),
                                preferred_element_type=jnp.float32)
  c_acc[...] += jnp.sum(ob, axis=0)[:, None]

  @pl.when(i == _N // _BN - 1)
  def _():
    pooled = p_acc[...] / jnp.maximum(c_acc[...], 1.0)
    out_ref[...] = (jnp.dot(pooled, wl_ref[...],
                            preferred_element_type=jnp.float32) + bl_ref[...])


def _tc_final(root, a0, a1, scn, we, batchf, wl, bl):
  return pl.pallas_call(
      _kfin_body,
      grid=(_N // _BN,),
      in_specs=[
          pl.BlockSpec((_BN, _H), lambda i: (i, 0)),
          pl.BlockSpec((_BN, _H), lambda i: (i, 0)),
          pl.BlockSpec((_BN, _H), lambda i: (i, 0)),
          pl.BlockSpec((_BN, 8), lambda i: (i, 0)),
          _full(we.shape),
          pl.BlockSpec((_BN, 1), lambda i: (i, 0)),
          _full(wl.shape), _full(bl.shape),
      ],
      out_specs=pl.BlockSpec((_G, _C), lambda i: (0, 0)),
      out_shape=jax.ShapeDtypeStruct((_G, _C), jnp.float32),
      scratch_shapes=[
          pltpu.VMEM((_G, _H), jnp.float32),
          pltpu.VMEM((_G, 1), jnp.float32),
      ],
  )(root, a0, a1, scn, we, batchf, wl, bl)


def kernel(x, edge_index, edge_attr, edge_type, batch,
           W1, b1, W2, b2,
           Wroot1, Wrel1, We1, bc1,
           Wroot2, Wrel2, We2, bc2,
           Wroot3, Wrel3, We3, bc3,
           Wroot4, Wrel4, We4, bc4,
           Wl, bl):
  src = edge_index[0]
  dst = edge_index[1]
  et = edge_type

  pad = _EPAD - _E
  src3 = jnp.pad(src + (et << 14), (0, pad),
                 constant_values=(2 << 14)).reshape(_NSUB, _EPW)
  dst3p = jnp.pad(dst, (0, pad)).reshape(_NSUB, _EPW)

  atr3 = jnp.pad(edge_attr[:, 0].astype(jnp.float32),
                 (0, pad)).reshape(_NSUB, _EPW)
  srcb, dstb, atrb, nchb = _sc_compact(src3, dst3p, atr3)

  stats = _sc_stats2(atrb, dstb, nchb).reshape(2, _NQ * _QR, 16)
  scn = jnp.concatenate([
      stats[0, :_N, 0:2], stats[1, :_N, 0:2],
      jnp.zeros((_N, 4), jnp.float32)], axis=1)

  x = x.astype(jnp.float32)
  y, root = _tc_encode_l1(x, W1, b1.reshape(1, -1), W2, b2.reshape(1, -1),
                          Wrel1[0], Wrel1[1], Wroot1, bc1.reshape(1, -1))

  def aggs(y):
    out = _sc_scatter(y.reshape(2 * _N, _H), srcb, dstb, nchb)
    a = out.reshape(2, _NQ * _QR, 128)
    return a[0, :_N], a[1, :_N]

  for Wroot, Wrel, We, bc in ((Wroot2, Wrel2, We1, bc2),
                              (Wroot3, Wrel3, We2, bc3),
                              (Wroot4, Wrel4, We3, bc4)):
    a0, a1 = aggs(y)
    y, root = _tc_mid(root, a0, a1, scn, We.reshape(1, -1),
                      Wrel[0], Wrel[1], Wroot, bc.reshape(1, -1))

  a0, a1 = aggs(y)
  batchf = batch.astype(jnp.float32).reshape(_N, 1)
  return _tc_final(root, a0, a1, scn, We4.reshape(1, -1),
                   batchf, Wl, bl.reshape(1, -1))


# speedup vs baseline: 2.7606x; 1.0004x over previous
import functools

import jax
import jax.numpy as jnp
from jax import lax
from jax.experimental import pallas as pl
from jax.experimental.pallas import tpu as pltpu
from jax.experimental.pallas import tpu_sc as plsc

_N = 10000
_E = 320000
_H = 128
_G = 64
_C = 10

_NSUB = 16
_CH = 128
_EPW = 20480
_NCH = _EPW // _CH
_EPAD = _NSUB * _EPW
_NQ = 5
_QR = 2048
_ACCQ = 2176
_QTRASH = 2100
_QSTR = _QR // _NSUB


def _zero_vmem(ref, rows, width):
  @pl.loop(0, rows)
  def _(r):
    @pl.loop(0, width // 16)
    def _(k):
      ref[r, pl.ds(k * 16, 16)] = jnp.zeros((16,), jnp.float32)


@functools.lru_cache(maxsize=None)
def _sc_kernels():
  mesh = plsc.VectorSubcoreMesh(core_axis_name="c", subcore_axis_name="s",
                                num_cores=2, num_subcores=_NSUB)
  params = pltpu.CompilerParams(use_tc_tiling_on_sc=False,
                                needs_layout_passes=False)

  @functools.partial(
      pl.kernel,
      out_type=[
          jax.ShapeDtypeStruct((2, _NQ, _NSUB, _NCH, _CH), jnp.int32),
          jax.ShapeDtypeStruct((2, _NQ, _NSUB, _NCH, _CH), jnp.int32),
          jax.ShapeDtypeStruct((2, _NQ, _NSUB, _NCH, _CH), jnp.float32),
          jax.ShapeDtypeStruct((2, _NQ, _NSUB, 16), jnp.int32),
      ],
      mesh=mesh,
      compiler_params=params,
      scratch_types=[
          pltpu.VMEM((_EPW,), jnp.int32),
          pltpu.VMEM((_EPW,), jnp.int32),
          pltpu.VMEM((_EPW,), jnp.float32),
          pltpu.VMEM((_NCH + 3, _CH), jnp.int32),
          pltpu.VMEM((_NCH + 3, _CH), jnp.int32),
          pltpu.VMEM((_NCH + 3, _CH), jnp.float32),
          pltpu.VMEM((16,), jnp.int32),
      ],
  )
  def sc_compact(src_hbm, dst_hbm, atr_hbm,
                 srcb_hbm, dstb_hbm, atrb_hbm, nch_hbm,
                 src_sl, dst_sl, atr_sl, sbuf, dbuf, abuf, cnt_v):
    c = lax.axis_index("c")
    s = lax.axis_index("s")
    pltpu.sync_copy(src_hbm.at[s], src_sl)
    pltpu.sync_copy(dst_hbm.at[s], dst_sl)
    pltpu.sync_copy(atr_hbm.at[s], atr_sl)
    lanes = lax.iota(jnp.int32, 16)
    srow = c * _N

    for q in range(_NQ):
      @plsc.parallel_loop(0, _EPW // 16, carry=jnp.int32(0))
      def o(i, o_in):
        li = 16 * i + lanes
        sc_ = plsc.load_gather(src_sl, [li])
        dv = plsc.load_gather(dst_sl, [li])
        tv = sc_ >> 14
        sv = sc_ & 16383
        m = (tv == c) & (dv >= _QR * q) & (dv < _QR * (q + 1))
        mi = m.astype(jnp.int32)
        idx = o_in + plsc.cumsum(mi) - 1
        av = plsc.load_gather(atr_sl, [li])
        plsc.store_scatter(sbuf, [idx >> 7, idx & 127], sv + srow, mask=m)
        plsc.store_scatter(dbuf, [idx >> 7, idx & 127], dv - _QR * q, mask=m)
        plsc.store_scatter(abuf, [idx >> 7, idx & 127], av, mask=m)
        return o_in + jnp.sum(mi)

      ones = jnp.full((16,), True)
      for t in range(_CH // 16 + 1):
        pidx = o + lanes + 16 * t
        plsc.store_scatter(sbuf, [pidx >> 7, pidx & 127],
                           jnp.full((16,), 0, jnp.int32) + srow, mask=ones)
        plsc.store_scatter(dbuf, [pidx >> 7, pidx & 127],
                           jnp.full((16,), _QTRASH, jnp.int32), mask=ones)
        plsc.store_scatter(abuf, [pidx >> 7, pidx & 127],
                           jnp.zeros((16,), jnp.float32), mask=ones)
      nch = (o + _CH - 1) // _CH

      pltpu.sync_copy(sbuf.at[pl.ds(0, _NCH)], srcb_hbm.at[c, q, s])
      pltpu.sync_copy(dbuf.at[pl.ds(0, _NCH)], dstb_hbm.at[c, q, s])
      pltpu.sync_copy(abuf.at[pl.ds(0, _NCH)], atrb_hbm.at[c, q, s])
      cnt_v[...] = jnp.zeros((16,), jnp.int32) + nch
      pltpu.sync_copy(cnt_v, nch_hbm.at[c, q, s])

  @functools.partial(
      pl.kernel,
      out_type=jax.ShapeDtypeStruct((2, _NQ, _QR, 128), jnp.float32),
      mesh=mesh,
      compiler_params=params,
      scratch_types=[
          pltpu.VMEM((_NCH, _CH), jnp.int32),
          pltpu.VMEM((_NCH, _CH), jnp.int32),
          pltpu.VMEM((3, _CH, 128), jnp.float32),
          pltpu.VMEM((_QSTR // 2, 128), jnp.float32),
          pltpu.VMEM((16,), jnp.int32),
          pltpu.VMEM_SHARED((_ACCQ, 128), jnp.float32),
          [pltpu.SemaphoreType.DMA] * 3,
          [pltpu.SemaphoreType.DMA] * 3,
      ],
  )
  def sc_scatter(y_hbm, srcb_hbm, dstb_hbm, nch_hbm, out_hbm,
                 src_v, dst_v, rows_v, z_v, cnt_v, acc_sh, gsems, ssems):
    c = lax.axis_index("c")
    s = lax.axis_index("s")
    _zero_vmem(z_v, _QSTR // 2, 128)
    base = s * _QSTR

    for q in range(_NQ):
      pltpu.sync_copy(nch_hbm.at[c, q, s], cnt_v)
      nch = jnp.max(cnt_v[...])
      for p in range(4):
        @pl.when(p * 40 < nch)
        def _():
          pltpu.sync_copy(srcb_hbm.at[c, q, s, pl.ds(p * 40, 40)],
                          src_v.at[pl.ds(p * 40, 40)])
          pltpu.sync_copy(dstb_hbm.at[c, q, s, pl.ds(p * 40, 40)],
                          dst_v.at[pl.ds(p * 40, 40)])
      for hh in range(2):
        pltpu.sync_copy(z_v, acc_sh.at[pl.ds(base + hh * (_QSTR // 2),
                                             _QSTR // 2)])
      plsc.subcore_barrier()

      for b in range(3):
        @pl.when(b < nch)
        def _():
          pltpu.async_copy(y_hbm.at[src_v.at[b]], rows_v.at[b], gsems[b])

      @pl.loop(0, (nch + 2) // 3)
      def _(jr):
        j = 3 * jr
        for b in range(3):
          @pl.when(j + b < nch)
          def _():
            pltpu.make_async_copy(y_hbm.at[src_v.at[0]], rows_v.at[b],
                                  gsems[b]).wait()
            pltpu.async_copy(rows_v.at[b], acc_sh.at[dst_v.at[j + b]],
                             ssems[b], add=True)
        for b in range(3):
          @pl.when(j + b + 3 < nch)
          def _():
            pltpu.make_async_copy(y_hbm.at[src_v.at[0]], rows_v.at[b],
                                  ssems[b]).wait()
            pltpu.async_copy(y_hbm.at[src_v.at[j + b + 3]], rows_v.at[b],
                             gsems[b])

      for b in range(3):
        @pl.when(b < jnp.minimum(nch, 3))
        def _():
          pltpu.make_async_copy(y_hbm.at[src_v.at[0]], rows_v.at[b],
                                ssems[b]).wait()

      plsc.subcore_barrier()
      pltpu.sync_copy(acc_sh.at[pl.ds(base, _QSTR)],
                      out_hbm.at[c, q, pl.ds(base, _QSTR)])

  @functools.partial(
      pl.kernel,
      out_type=jax.ShapeDtypeStruct((2, _NQ, _QR, 16), jnp.float32),
      mesh=mesh,
      compiler_params=params,
      scratch_types=[
          pltpu.VMEM((_NCH, _CH), jnp.int32),
          pltpu.VMEM((_NCH, _CH), jnp.float32),
          pltpu.VMEM((_CH, 16), jnp.float32),
          pltpu.VMEM((_QSTR // 2, 16), jnp.float32),
          pltpu.VMEM((16,), jnp.int32),
          pltpu.VMEM_SHARED((_ACCQ, 16), jnp.float32),
      ],
  )
  def sc_stats2(atrb_hbm, dstb_hbm, nch_hbm, out_hbm,
                dst_v, atr_v, rowb, z_v, cnt_v, acc_sh):
    c = lax.axis_index("c")
    s = lax.axis_index("s")
    lanes = lax.iota(jnp.int32, 16)
    _zero_vmem(z_v, _QSTR // 2, 16)
    _zero_vmem(rowb, _CH, 16)
    for k in range(_CH // 16):
      plsc.store_scatter(rowb, [k * 16 + lanes, jnp.full((16,), 1, jnp.int32)],
                         jnp.full((16,), 1.0, jnp.float32))
    base = s * _QSTR

    for q in range(_NQ):
      pltpu.sync_copy(nch_hbm.at[c, q, s], cnt_v)
      nch = jnp.max(cnt_v[...])
      for p in range(4):
        @pl.when(p * 40 < nch)
        def _():
          pltpu.sync_copy(atrb_hbm.at[c, q, s, pl.ds(p * 40, 40)],
                          atr_v.at[pl.ds(p * 40, 40)])
          pltpu.sync_copy(dstb_hbm.at[c, q, s, pl.ds(p * 40, 40)],
                          dst_v.at[pl.ds(p * 40, 40)])
      for hh in range(2):
        pltpu.sync_copy(z_v, acc_sh.at[pl.ds(base + hh * (_QSTR // 2),
                                             _QSTR // 2)])
      plsc.subcore_barrier()

      @pl.loop(0, nch)
      def _(j):
        for k in range(_CH // 16):
          av = plsc.load_gather(atr_v, [jnp.full((16,), 0, jnp.int32) + j,
                                        k * 16 + lanes])
          plsc.store_scatter(rowb, [k * 16 + lanes,
                                    jnp.zeros((16,), jnp.int32)], av)
        pltpu.sync_copy(rowb, acc_sh.at[dst_v.at[j]], add=True)

      plsc.subcore_barrier()
      pltpu.sync_copy(acc_sh.at[pl.ds(base, _QSTR)],
                      out_hbm.at[c, q, pl.ds(base, _QSTR)])

  return sc_compact, sc_scatter, sc_stats2


def _sc_compact(src3, dst3p, atr3):
  return _sc_kernels()[0](src3, dst3p, atr3)


def _sc_scatter(y, srcb, dstb, nchb):
  return _sc_kernels()[1](y, srcb, dstb, nchb)


def _sc_stats2(atrb, dstb, nchb):
  return _sc_kernels()[2](atrb, dstb, nchb)


_BN = 1000


def _full(shape):
  return pl.BlockSpec(shape, lambda i: (0,) * len(shape))


def _k1_body(x_ref, w1_ref, b1_ref, w2_ref, b2_ref,
             wr0_ref, wr1_ref, wroot_ref, bc_ref, y_ref, root_ref):
  x = x_ref[...]
  t = jnp.dot(x, w1_ref[...], preferred_element_type=jnp.float32) + b1_ref[...]
  h = jnp.dot(t, w2_ref[...], preferred_element_type=jnp.float32) + b2_ref[...]
  y_ref[0] = jnp.dot(h, wr0_ref[...], preferred_element_type=jnp.float32)
  y_ref[1] = jnp.dot(h, wr1_ref[...], preferred_element_type=jnp.float32)
  root_ref[...] = (jnp.dot(h, wroot_ref[...], preferred_element_type=jnp.float32)
                   + bc_ref[...])


def _tc_encode_l1(x, w1, b1, w2, b2, wr0, wr1, wroot, bc):
  d_in = x.shape[1]
  d_h = w2.shape[1]
  return pl.pallas_call(
      _k1_body,
      grid=(_N // _BN,),
      in_specs=[
          pl.BlockSpec((_BN, d_in), lambda i: (i, 0)),
          _full(w1.shape), _full(b1.shape), _full(w2.shape), _full(b2.shape),
          _full((d_h, _H)), _full((d_h, _H)), _full((d_h, _H)), _full(bc.shape),
      ],
      out_specs=[
          pl.BlockSpec((2, _BN, _H), lambda i: (0, i, 0)),
          pl.BlockSpec((_BN, _H), lambda i: (i, 0)),
      ],
      out_shape=[
          jax.ShapeDtypeStruct((2, _N, _H), jnp.float32),
          jax.ShapeDtypeStruct((_N, _H), jnp.float32),
      ],
  )(x, w1, b1, w2, b2, wr0, wr1, wroot, bc)


def _epilogue(root_ref, a0_ref, a1_ref, scn_ref, we_ref):
  s0 = scn_ref[:, 0:1]
  c0 = scn_ref[:, 1:2]
  s1 = scn_ref[:, 2:3]
  c1 = scn_ref[:, 3:4]
  we = we_ref[...]
  t0 = (a0_ref[...] + s0 * we) / jnp.maximum(c0, 1.0)
  t1 = (a1_ref[...] + s1 * we) / jnp.maximum(c1, 1.0)
  h = root_ref[...] + t0 + t1
  return jnp.where(h > 0.0, h, jnp.exp(jnp.minimum(h, 0.0)) - 1.0)


def _kmid_body(root_ref, a0_ref, a1_ref, scn_ref, we_ref,
               wr0_ref, wr1_ref, wroot_ref, bc_ref, y_ref, rootn_ref):
  h = _epilogue(root_ref, a0_ref, a1_ref, scn_ref, we_ref)
  y_ref[0] = jnp.dot(h, wr0_ref[...], preferred_element_type=jnp.float32)
  y_ref[1] = jnp.dot(h, wr1_ref[...], preferred_element_type=jnp.float32)
  rootn_ref[...] = (jnp.dot(h, wroot_ref[...],
                            preferred_element_type=jnp.float32) + bc_ref[...])


def _tc_mid(root, a0, a1, scn, we, wr0, wr1, wroot, bc):
  return pl.pallas_call(
      _kmid_body,
      grid=(_N // _BN,),
      in_specs=[
          pl.BlockSpec((_BN, _H), lambda i: (i, 0)),
          pl.BlockSpec((_BN, _H), lambda i: (i, 0)),
          pl.BlockSpec((_BN, _H), lambda i: (i, 0)),
          pl.BlockSpec((_BN, 8), lambda i: (i, 0)),
          _full(we.shape),
          _full((_H, _H)), _full((_H, _H)), _full((_H, _H)), _full(bc.shape),
      ],
      out_specs=[
          pl.BlockSpec((2, _BN, _H), lambda i: (0, i, 0)),
          pl.BlockSpec((_BN, _H), lambda i: (i, 0)),
      ],
      out_shape=[
          jax.ShapeDtypeStruct((2, _N, _H), jnp.float32),
          jax.ShapeDtypeStruct((_N, _H), jnp.float32),
      ],
  )(root, a0, a1, scn, we, wr0, wr1, wroot, bc)


def _kfin_body(root_ref, a0_ref, a1_ref, scn_ref, we_ref,
               batch_ref, wl_ref, bl_ref, out_ref, p_acc, c_acc):
  i = pl.program_id(0)

  @pl.when(i == 0)
  def _():
    p_acc[...] = jnp.zeros_like(p_acc)
    c_acc[...] = jnp.zeros_like(c_acc)

  h = _epilogue(root_ref, a0_ref, a1_ref, scn_ref, we_ref)
  bf = batch_ref[...]
  gids = lax.broadcasted_iota(jnp.int32, (_BN, _G), 1).astype(jnp.float32)
  ob = (bf == gids).astype(jnp.float32)
  p_acc[...] += lax.dot_general(ob, h, (((0,), (0,)), ((), ())),
                                preferred_element_type=jnp.float32)
  c_acc[...] += jnp.sum(ob, axis=0)[:, None]

  @pl.when(i == _N // _BN - 1)
  def _():
    pooled = p_acc[...] / jnp.maximum(c_acc[...], 1.0)
    out_ref[...] = (jnp.dot(pooled, wl_ref[...],
                            preferred_element_type=jnp.float32) + bl_ref[...])


def _tc_final(root, a0, a1, scn, we, batchf, wl, bl):
  return pl.pallas_call(
      _kfin_body,
      grid=(_N // _BN,),
      in_specs=[
          pl.BlockSpec((_BN, _H), lambda i: (i, 0)),
          pl.BlockSpec((_BN, _H), lambda i: (i, 0)),
          pl.BlockSpec((_BN, _H), lambda i: (i, 0)),
          pl.BlockSpec((_BN, 8), lambda i: (i, 0)),
          _full(we.shape),
          pl.BlockSpec((_BN, 1), lambda i: (i, 0)),
          _full(wl.shape), _full(bl.shape),
      ],
      out_specs=pl.BlockSpec((_G, _C), lambda i: (0, 0)),
      out_shape=jax.ShapeDtypeStruct((_G, _C), jnp.float32),
      scratch_shapes=[
          pltpu.VMEM((_G, _H), jnp.float32),
          pltpu.VMEM((_G, 1), jnp.float32),
      ],
  )(root, a0, a1, scn, we, batchf, wl, bl)


def kernel(x, edge_index, edge_attr, edge_type, batch,
           W1, b1, W2, b2,
           Wroot1, Wrel1, We1, bc1,
           Wroot2, Wrel2, We2, bc2,
           Wroot3, Wrel3, We3, bc3,
           Wroot4, Wrel4, We4, bc4,
           Wl, bl):
  src = edge_index[0]
  dst = edge_index[1]
  et = edge_type

  pad = _EPAD - _E
  src3 = jnp.pad(src + (et << 14), (0, pad),
                 constant_values=(2 << 14)).reshape(_NSUB, _EPW)
  dst3p = jnp.pad(dst, (0, pad)).reshape(_NSUB, _EPW)

  atr3 = jnp.pad(edge_attr[:, 0].astype(jnp.float32),
                 (0, pad)).reshape(_NSUB, _EPW)
  srcb, dstb, atrb, nchb = _sc_compact(src3, dst3p, atr3)

  stats = _sc_stats2(atrb, dstb, nchb).reshape(2, _NQ * _QR, 16)
  scn = jnp.concatenate([
      stats[0, :_N, 0:2], stats[1, :_N, 0:2],
      jnp.zeros((_N, 4), jnp.float32)], axis=1)

  x = x.astype(jnp.float32)
  y, root = _tc_encode_l1(x, W1, b1.reshape(1, -1), W2, b2.reshape(1, -1),
                          Wrel1[0], Wrel1[1], Wroot1, bc1.reshape(1, -1))

  def aggs(y):
    out = _sc_scatter(y.reshape(2 * _N, _H), srcb, dstb, nchb)
    a = out.reshape(2, _NQ * _QR, 128)
    return a[0, :_N], a[1, :_N]

  for Wroot, Wrel, We, bc in ((Wroot2, Wrel2, We1, bc2),
                              (Wroot3, Wrel3, We2, bc3),
                              (Wroot4, Wrel4, We3, bc4)):
    a0, a1 = aggs(y)
    y, root = _tc_mid(root, a0, a1, scn, We.reshape(1, -1),
                      Wrel[0], Wrel[1], Wroot, bc.reshape(1, -1))

  a0, a1 = aggs(y)
  batchf = batch.astype(jnp.float32).reshape(_N, 1)
  return _tc_final(root, a0, a1, scn, We4.reshape(1, -1),
                   batchf, Wl, bl.reshape(1, -1))
